# Initial kernel scaffold; baseline (speedup 1.0000x reference)
import jax
import jax.numpy as jnp
from jax.experimental import pallas as pl


def kernel(x, edge_index, batch, W_emb, b_emb, Wg, bg, ln1w, ln1b, Wf, bf, ln2w, ln2b):
    raise NotImplementedError("write your pallas kernel here")


# speedup vs baseline: 13.8884x; 13.8884x over previous
import functools

import jax
import jax.numpy as jnp
from jax import lax
from jax.experimental import pallas as pl
from jax.experimental.pallas import tpu as pltpu
from jax.experimental.pallas import tpu_sc as plsc

N = 10000
E = 320000
C = 128
L = 2

BN = 1000
NB = N // BN

NSC = 2
NSUB = 16
NW = NSC * NSUB
K = 128
NCH = E // K
CH_PER_W = -(-NCH // NW)
RPS = 624
RCH = 208
NRCH = RPS // RCH
TAIL0 = NSUB * RPS
TAIL = N - TAIL0

_HIGHEST = jax.lax.Precision.HIGHEST


def _sc_mesh():
    return plsc.VectorSubcoreMesh(core_axis_name="c", subcore_axis_name="s")


def _sc_deg(dst):
    @functools.partial(
        pl.kernel,
        out_type=jax.ShapeDtypeStruct((2 * N, C), jnp.float32),
        mesh=_sc_mesh(),
        scratch_types=[
            pltpu.VMEM((K,), jnp.int32),
            pltpu.VMEM((K, C), jnp.float32),
            pltpu.VMEM((RCH, C), jnp.float32),
            pltpu.VMEM_SHARED((N, C), jnp.float32),
        ],
    )
    def run(dst_hbm, out_hbm, idx_v, ones_v, zb, acc):
        c = lax.axis_index("c")
        s = lax.axis_index("s")
        wid = c * NSUB + s

        def fill_ones(i, carry):
            for j in range(C // 16):
                ones_v[i, pl.ds(j * 16, 16)] = jnp.full((16,), 1.0,
                                                        jnp.float32)
            return carry

        lax.fori_loop(0, K, fill_ones, 0)

        def fill_zero(i, carry):
            for j in range(C // 16):
                zb[i, pl.ds(j * 16, 16)] = jnp.zeros((16,), jnp.float32)
            return carry

        lax.fori_loop(0, RCH, fill_zero, 0)
        for k in range(NRCH):
            pltpu.sync_copy(zb, acc.at[pl.ds(s * RPS + k * RCH, RCH)])

        @pl.when(s == 0)
        def _():
            pltpu.sync_copy(zb.at[pl.ds(0, TAIL)], acc.at[pl.ds(TAIL0, TAIL)])

        plsc.subcore_barrier()

        def chunk(i, carry):
            cid = wid + i * NW

            @pl.when(cid < NCH)
            def _():
                pltpu.sync_copy(dst_hbm.at[pl.ds(cid * K, K)], idx_v)
                pltpu.sync_copy(ones_v, acc.at[idx_v], add=True)

            return carry

        lax.fori_loop(0, CH_PER_W, chunk, 0)
        plsc.subcore_barrier()
        for k in range(NRCH):
            r0 = s * RPS + k * RCH
            pltpu.sync_copy(acc.at[pl.ds(r0, RCH)],
                            out_hbm.at[pl.ds(c * N + r0, RCH)])

        @pl.when(s == 0)
        def _():
            pltpu.sync_copy(acc.at[pl.ds(TAIL0, TAIL)],
                            out_hbm.at[pl.ds(c * N + TAIL0, TAIL)])

    return run(dst)


def _sc_agg(src, dst, hs):
    @functools.partial(
        pl.kernel,
        out_type=jax.ShapeDtypeStruct((2 * N, C), jnp.float32),
        mesh=_sc_mesh(),
        scratch_types=[
            pltpu.VMEM((K,), jnp.int32),
            pltpu.VMEM((K,), jnp.int32),
            pltpu.VMEM((K, C), jnp.float32),
            pltpu.VMEM((RCH, C), jnp.float32),
            pltpu.VMEM_SHARED((N, C), jnp.float32),
            pltpu.SemaphoreType.DMA,
        ],
    )
    def run(src_hbm, dst_hbm, hs_hbm, out_hbm, idx_s, idx_d, rows_v, zb, acc,
            sem):
        c = lax.axis_index("c")
        s = lax.axis_index("s")
        wid = c * NSUB + s

        def fill_zero(i, carry):
            for j in range(C // 16):
                zb[i, pl.ds(j * 16, 16)] = jnp.zeros((16,), jnp.float32)
            return carry

        lax.fori_loop(0, RCH, fill_zero, 0)
        for k in range(NRCH):
            pltpu.sync_copy(zb, acc.at[pl.ds(s * RPS + k * RCH, RCH)])

        @pl.when(s == 0)
        def _():
            pltpu.sync_copy(zb.at[pl.ds(0, TAIL)], acc.at[pl.ds(TAIL0, TAIL)])

        plsc.subcore_barrier()

        def chunk(i, carry):
            cid = wid + i * NW

            @pl.when(cid < NCH)
            def _():
                pltpu.sync_copy(src_hbm.at[pl.ds(cid * K, K)], idx_s)
                pltpu.sync_copy(dst_hbm.at[pl.ds(cid * K, K)], idx_d)
                pltpu.async_copy(hs_hbm.at[idx_s], rows_v, sem).wait()
                pltpu.sync_copy(rows_v, acc.at[idx_d], add=True)

            return carry

        lax.fori_loop(0, CH_PER_W, chunk, 0)
        plsc.subcore_barrier()
        for k in range(NRCH):
            r0 = s * RPS + k * RCH
            pltpu.sync_copy(acc.at[pl.ds(r0, RCH)],
                            out_hbm.at[pl.ds(c * N + r0, RCH)])

        @pl.when(s == 0)
        def _():
            pltpu.sync_copy(acc.at[pl.ds(TAIL0, TAIL)],
                            out_hbm.at[pl.ds(c * N + TAIL0, TAIL)])

    return run(src, dst, hs)


def _dinv_block(degp_ref):
    d = degp_ref[0, :, 0:1] + degp_ref[1, :, 0:1] + 1.0
    return lax.rsqrt(d)


def _row_spec():
    return pl.BlockSpec((BN, C), lambda j: (j, 0))


def _full_spec(shape):
    return pl.BlockSpec(shape, lambda j: tuple(0 for _ in shape))


def _degp_spec():
    return pl.BlockSpec((2, BN, C), lambda j: (0, j, 0))


def _smem_spec():
    return pl.BlockSpec(memory_space=pltpu.SMEM)


def _tc_embed_body(x_ref, we_ref, be_ref, wg_ref, degp_ref,
                   h_ref, hs_ref, gb_ref, nb_ref, gacc, nacc):
    j = pl.program_id(0)
    h = jnp.dot(x_ref[...], we_ref[...], precision=_HIGHEST,
                preferred_element_type=jnp.float32) + be_ref[...]
    h_ref[...] = h
    hl = jnp.dot(h, wg_ref[...], precision=_HIGHEST,
                 preferred_element_type=jnp.float32)
    hs_ref[...] = hl * _dinv_block(degp_ref)

    @pl.when(j == 0)
    def _():
        gacc[...] = jnp.zeros_like(gacc)
        nacc[...] = jnp.zeros_like(nacc)

    gacc[...] += jnp.broadcast_to(jnp.sum(h, axis=0, keepdims=True), (8, C))
    nacc[...] += lax.dot_general(h, h, (((0,), (0,)), ((), ())),
                                 precision=_HIGHEST,
                                 preferred_element_type=jnp.float32)

    @pl.when(j == NB - 1)
    def _():
        gb_ref[...] = gacc[...]
        nb_ref[...] = nacc[...]


def _tc_embed(x, W_emb, b_emb, Wg0, degp):
    return pl.pallas_call(
        _tc_embed_body,
        grid=(NB,),
        in_specs=[_row_spec(), _full_spec((C, C)), _full_spec((1, C)),
                  _full_spec((C, C)), _degp_spec()],
        out_specs=[_row_spec(), _row_spec(), _full_spec((8, C)),
                   _full_spec((C, C))],
        out_shape=[jax.ShapeDtypeStruct((N, C), jnp.float32),
                   jax.ShapeDtypeStruct((N, C), jnp.float32),
                   jax.ShapeDtypeStruct((8, C), jnp.float32),
                   jax.ShapeDtypeStruct((C, C), jnp.float32)],
        scratch_shapes=[pltpu.VMEM((8, C), jnp.float32),
                        pltpu.VMEM((C, C), jnp.float32)],
    )(x, W_emb, b_emb, Wg0, degp)


def _tc_msum_body(aggp_ref, hs_ref, degp_ref, bg_ref,
                  m_ref, s1_ref, s2_ref, a1, a2):
    j = pl.program_id(0)
    m = (aggp_ref[0] + aggp_ref[1] + hs_ref[...]) * _dinv_block(degp_ref) \
        + bg_ref[...]
    m_ref[...] = m

    @pl.when(j == 0)
    def _():
        a1[...] = jnp.zeros_like(a1)
        a2[...] = jnp.zeros_like(a2)

    a1[...] += jnp.broadcast_to(jnp.sum(m, axis=0, keepdims=True), (8, C))
    a2[...] += jnp.broadcast_to(jnp.sum(m * m, axis=0, keepdims=True), (8, C))

    @pl.when(j == NB - 1)
    def _():
        s1_ref[...] = jnp.full((8, C), jnp.sum(a1[0:1, :]), jnp.float32)
        s2_ref[...] = jnp.full((8, C), jnp.sum(a2[0:1, :]), jnp.float32)


def _tc_msum(aggp, hs, degp, bg):
    return pl.pallas_call(
        _tc_msum_body,
        grid=(NB,),
        in_specs=[pl.BlockSpec((2, BN, C), lambda j: (0, j, 0)), _row_spec(),
                  _degp_spec(), _full_spec((1, C))],
        out_specs=[_row_spec(), _full_spec((8, C)), _full_spec((8, C))],
        out_shape=[jax.ShapeDtypeStruct((N, C), jnp.float32),
                   jax.ShapeDtypeStruct((8, C), jnp.float32),
                   jax.ShapeDtypeStruct((8, C), jnp.float32)],
        scratch_shapes=[pltpu.VMEM((8, C), jnp.float32),
                        pltpu.VMEM((8, C), jnp.float32)],
    )(aggp, hs, degp, bg)


_INV_NC = 1.0 / (N * C)
_EPS = 1e-5


def _ln_scale(s1_ref, s2_ref):
    mean = s1_ref[0:1, 0:1] * _INV_NC
    var = s2_ref[0:1, 0:1] * _INV_NC - mean * mean
    rstd = lax.rsqrt(var + _EPS)
    return mean, rstd


def _tc_ffn_body(m_ref, s1_ref, s2_ref, hp_ref, wf_ref, bf_ref,
                 lnw_ref, lnb_ref, f_ref, t1_ref, t2_ref, a1, a2):
    j = pl.program_id(0)
    mean, rstd = _ln_scale(s1_ref, s2_ref)
    mhat = (m_ref[...] - mean) * rstd * lnw_ref[0, 0] + lnb_ref[0, 0]
    h = hp_ref[...] + jnp.maximum(mhat, 0.0)
    f = jnp.dot(h, wf_ref[...], precision=_HIGHEST,
                preferred_element_type=jnp.float32) + bf_ref[...]
    f_ref[...] = f

    @pl.when(j == 0)
    def _():
        a1[...] = jnp.zeros_like(a1)
        a2[...] = jnp.zeros_like(a2)

    a1[...] += jnp.broadcast_to(jnp.sum(f, axis=0, keepdims=True), (8, C))
    a2[...] += jnp.broadcast_to(jnp.sum(f * f, axis=0, keepdims=True), (8, C))

    @pl.when(j == NB - 1)
    def _():
        t1_ref[...] = jnp.full((8, C), jnp.sum(a1[0:1, :]), jnp.float32)
        t2_ref[...] = jnp.full((8, C), jnp.sum(a2[0:1, :]), jnp.float32)


def _tc_ffn(m, s1, s2, hprev, Wf_i, bf_i, lnw, lnb):
    return pl.pallas_call(
        _tc_ffn_body,
        grid=(NB,),
        in_specs=[_row_spec(), _full_spec((8, C)), _full_spec((8, C)),
                  _row_spec(), _full_spec((C, C)), _full_spec((1, C)),
                  _smem_spec(), _smem_spec()],
        out_specs=[_row_spec(), _full_spec((8, C)), _full_spec((8, C))],
        out_shape=[jax.ShapeDtypeStruct((N, C), jnp.float32),
                   jax.ShapeDtypeStruct((8, C), jnp.float32),
                   jax.ShapeDtypeStruct((8, C), jnp.float32)],
        scratch_shapes=[pltpu.VMEM((8, C), jnp.float32),
                        pltpu.VMEM((8, C), jnp.float32)],
    )(m, s1, s2, hprev, Wf_i, bf_i, lnw, lnb)


def _tc_out_body(f_ref, s1_ref, s2_ref, lnw_ref, lnb_ref, degp_ref, wg_ref,
                 h_ref, hs_ref, gb_ref, nb_ref, gacc, nacc):
    j = pl.program_id(0)
    mean, rstd = _ln_scale(s1_ref, s2_ref)
    fhat = (f_ref[...] - mean) * rstd * lnw_ref[0, 0] + lnb_ref[0, 0]
    h = jnp.maximum(fhat, 0.0)
    h_ref[...] = h
    hl = jnp.dot(h, wg_ref[...], precision=_HIGHEST,
                 preferred_element_type=jnp.float32)
    hs_ref[...] = hl * _dinv_block(degp_ref)

    @pl.when(j == 0)
    def _():
        gacc[...] = jnp.zeros_like(gacc)
        nacc[...] = jnp.zeros_like(nacc)

    gacc[...] += jnp.broadcast_to(jnp.sum(h, axis=0, keepdims=True), (8, C))
    nacc[...] += lax.dot_general(h, h, (((0,), (0,)), ((), ())),
                                 precision=_HIGHEST,
                                 preferred_element_type=jnp.float32)

    @pl.when(j == NB - 1)
    def _():
        gb_ref[...] = gacc[...]
        nb_ref[...] = nacc[...]


def _tc_out(f, s1, s2, lnw, lnb, degp, Wg_next):
    return pl.pallas_call(
        _tc_out_body,
        grid=(NB,),
        in_specs=[_row_spec(), _full_spec((8, C)), _full_spec((8, C)),
                  _smem_spec(), _smem_spec(), _degp_spec(),
                  _full_spec((C, C))],
        out_specs=[_row_spec(), _row_spec(), _full_spec((8, C)),
                   _full_spec((C, C))],
        out_shape=[jax.ShapeDtypeStruct((N, C), jnp.float32),
                   jax.ShapeDtypeStruct((N, C), jnp.float32),
                   jax.ShapeDtypeStruct((8, C), jnp.float32),
                   jax.ShapeDtypeStruct((C, C), jnp.float32)],
        scratch_shapes=[pltpu.VMEM((8, C), jnp.float32),
                        pltpu.VMEM((C, C), jnp.float32)],
    )(f, s1, s2, lnw, lnb, degp, Wg_next)


def _tc_out_last_body(f_ref, s1_ref, s2_ref, lnw_ref, lnb_ref,
                      gb_ref, nb_ref, gacc, nacc):
    j = pl.program_id(0)
    mean, rstd = _ln_scale(s1_ref, s2_ref)
    fhat = (f_ref[...] - mean) * rstd * lnw_ref[0, 0] + lnb_ref[0, 0]
    h = jnp.maximum(fhat, 0.0)

    @pl.when(j == 0)
    def _():
        gacc[...] = jnp.zeros_like(gacc)
        nacc[...] = jnp.zeros_like(nacc)

    gacc[...] += jnp.broadcast_to(jnp.sum(h, axis=0, keepdims=True), (8, C))
    nacc[...] += lax.dot_general(h, h, (((0,), (0,)), ((), ())),
                                 precision=_HIGHEST,
                                 preferred_element_type=jnp.float32)

    @pl.when(j == NB - 1)
    def _():
        gb_ref[...] = gacc[...]
        nb_ref[...] = nacc[...]


def _tc_out_last(f, s1, s2, lnw, lnb):
    return pl.pallas_call(
        _tc_out_last_body,
        grid=(NB,),
        in_specs=[_row_spec(), _full_spec((8, C)), _full_spec((8, C)),
                  _smem_spec(), _smem_spec()],
        out_specs=[_full_spec((8, C)), _full_spec((C, C))],
        out_shape=[jax.ShapeDtypeStruct((8, C), jnp.float32),
                   jax.ShapeDtypeStruct((C, C), jnp.float32)],
        scratch_shapes=[pltpu.VMEM((8, C), jnp.float32),
                        pltpu.VMEM((C, C), jnp.float32)],
    )(f, s1, s2, lnw, lnb)


def kernel(x, edge_index, batch, W_emb, b_emb, Wg, bg, ln1w, ln1b,
           Wf, bf, ln2w, ln2b):
    src = edge_index[0]
    dst = edge_index[1]
    degp = _sc_deg(dst).reshape(2, N, C)

    h0, hs0, gb0, nb0 = _tc_embed(x, W_emb, b_emb[None, :], Wg[0], degp)

    gbs, nbs = [gb0], [nb0]
    h, hs = h0, hs0
    for i in range(L):
        aggp = _sc_agg(src, dst, hs).reshape(2, N, C)
        m, s1, s2 = _tc_msum(aggp, hs, degp, bg[i][None, :])
        f, t1, t2 = _tc_ffn(m, s1, s2, h, Wf[i], bf[i][None, :],
                            ln1w[i].reshape(1, 1), ln1b[i].reshape(1, 1))
        if i + 1 < L:
            h, hs, gb_i, nb_i = _tc_out(f, t1, t2, ln2w[i].reshape(1, 1),
                                        ln2b[i].reshape(1, 1), degp, Wg[i + 1])
        else:
            gb_i, nb_i = _tc_out_last(f, t1, t2, ln2w[i].reshape(1, 1),
                                      ln2b[i].reshape(1, 1))
        gbs.append(gb_i)
        nbs.append(nb_i)

    gb_out = jnp.concatenate([g[0:1] for g in gbs], axis=-1)
    nb_out = jnp.stack(nbs, axis=0)[None]
    return gb_out, nb_out


# speedup vs baseline: 21.7406x; 1.5654x over previous
import functools

import jax
import jax.numpy as jnp
from jax import lax
from jax.experimental import pallas as pl
from jax.experimental.pallas import tpu as pltpu
from jax.experimental.pallas import tpu_sc as plsc

N = 10000
E = 320000
C = 128
L = 2

BN = 1000
NB = N // BN

NSC = 2
NSUB = 16
NW = NSC * NSUB
K = 128
NCH = E // K
CH_PER_W = -(-NCH // NW)
RPS = 624
RCH = 104
NRCH = RPS // RCH
TAIL0 = NSUB * RPS
TAIL = N - TAIL0

_HIGHEST = jax.lax.Precision.HIGHEST


def _sc_mesh():
    return plsc.VectorSubcoreMesh(core_axis_name="c", subcore_axis_name="s")


def _sc_deg(dst):
    @functools.partial(
        pl.kernel,
        out_type=jax.ShapeDtypeStruct((2 * N, C), jnp.float32),
        mesh=_sc_mesh(),
        scratch_types=[
            pltpu.VMEM((K,), jnp.int32),
            pltpu.VMEM((K, C), jnp.float32),
            pltpu.VMEM((RCH, C), jnp.float32),
            pltpu.VMEM_SHARED((N, C), jnp.float32),
        ],
    )
    def run(dst_hbm, out_hbm, idx_v, ones_v, zb, acc):
        c = lax.axis_index("c")
        s = lax.axis_index("s")
        wid = c * NSUB + s

        def fill_ones(i, carry):
            for j in range(C // 16):
                ones_v[i, pl.ds(j * 16, 16)] = jnp.full((16,), 1.0,
                                                        jnp.float32)
            return carry

        lax.fori_loop(0, K, fill_ones, 0)

        def fill_zero(i, carry):
            for j in range(C // 16):
                zb[i, pl.ds(j * 16, 16)] = jnp.zeros((16,), jnp.float32)
            return carry

        lax.fori_loop(0, RCH, fill_zero, 0)
        for k in range(NRCH):
            pltpu.sync_copy(zb, acc.at[pl.ds(s * RPS + k * RCH, RCH)])

        @pl.when(s == 0)
        def _():
            pltpu.sync_copy(zb.at[pl.ds(0, TAIL)], acc.at[pl.ds(TAIL0, TAIL)])

        plsc.subcore_barrier()

        def chunk(i, carry):
            cid = wid + i * NW

            @pl.when(cid < NCH)
            def _():
                pltpu.sync_copy(dst_hbm.at[pl.ds(cid * K, K)], idx_v)
                pltpu.sync_copy(ones_v, acc.at[idx_v], add=True)

            return carry

        lax.fori_loop(0, CH_PER_W, chunk, 0)
        plsc.subcore_barrier()
        for k in range(NRCH):
            r0 = s * RPS + k * RCH
            pltpu.sync_copy(acc.at[pl.ds(r0, RCH)],
                            out_hbm.at[pl.ds(c * N + r0, RCH)])

        @pl.when(s == 0)
        def _():
            pltpu.sync_copy(acc.at[pl.ds(TAIL0, TAIL)],
                            out_hbm.at[pl.ds(c * N + TAIL0, TAIL)])

    return run(dst)


CHU = 80


def _sc_agg(src, dst, hs):
    @functools.partial(
        pl.kernel,
        out_type=jax.ShapeDtypeStruct((2 * N, C), jnp.float32),
        mesh=_sc_mesh(),
        scratch_types=(
            [pltpu.VMEM((K,), jnp.int32) for _ in range(8)]
            + [pltpu.VMEM((K, C), jnp.float32) for _ in range(2)]
            + [pltpu.VMEM((RCH, C), jnp.float32),
               pltpu.VMEM_SHARED((N, C), jnp.float32)]
            + [pltpu.SemaphoreType.DMA for _ in range(10)]
        ),
    )
    def run(src_hbm, dst_hbm, hs_hbm, out_hbm,
            is0, is1, is2, is3, id0, id1, id2, id3, rows0, rows1, zb, acc,
            gis0, gis1, gis2, gis3, gid0, gid1, gid2, gid3, gg0, gg1):
        isl = [is0, is1, is2, is3]
        idl = [id0, id1, id2, id3]
        rows = [rows0, rows1]
        sis = [gis0, gis1, gis2, gis3]
        sid = [gid0, gid1, gid2, gid3]
        sg = [gg0, gg1]
        c = lax.axis_index("c")
        s = lax.axis_index("s")
        wid = c * NSUB + s

        def fill_zero(i, carry):
            for j in range(C // 16):
                zb[i, pl.ds(j * 16, 16)] = jnp.zeros((16,), jnp.float32)
            return carry

        lax.fori_loop(0, RCH, fill_zero, 0)
        for k in range(NRCH):
            pltpu.sync_copy(zb, acc.at[pl.ds(s * RPS + k * RCH, RCH)])

        @pl.when(s == 0)
        def _():
            pltpu.sync_copy(zb.at[pl.ds(0, TAIL)], acc.at[pl.ds(TAIL0, TAIL)])

        plsc.subcore_barrier()

        def base_of(i):
            cid = wid + i * NW
            return jnp.where(cid < NCH, cid * K, 0), cid < NCH

        def issue_idx(i, slot):
            base, _ = base_of(i)
            pltpu.make_async_copy(src_hbm.at[pl.ds(base, K)], isl[slot],
                                  sis[slot]).start()
            pltpu.make_async_copy(dst_hbm.at[pl.ds(base, K)], idl[slot],
                                  sid[slot]).start()

        def wait_idx_s(slot):
            pltpu.make_async_copy(src_hbm.at[pl.ds(0, K)], isl[slot],
                                  sis[slot]).wait()

        def finish_chunk(i, slot, rb):
            pltpu.make_async_copy(hs_hbm.at[isl[slot]], rows[rb],
                                  sg[rb]).wait()
            pltpu.make_async_copy(dst_hbm.at[pl.ds(0, K)], idl[slot],
                                  sid[slot]).wait()
            _, valid = base_of(i)

            @pl.when(valid)
            def _():
                pltpu.sync_copy(rows[rb], acc.at[idl[slot]], add=True)

        for u in range(3):
            issue_idx(u, u)

        def step(p, carry):
            for u in range(4):
                i = 4 * p + u
                wait_idx_s(u)
                pltpu.make_async_copy(hs_hbm.at[isl[u]], rows[u % 2],
                                      sg[u % 2]).start()
                @pl.when(i >= 1)
                def _(u=u, i=i):
                    finish_chunk(i - 1, (u - 1) % 4, (u - 1) % 2)

                @pl.when(i + 3 < CHU)
                def _(u=u, i=i):
                    issue_idx(i + 3, (u + 3) % 4)

            return carry

        lax.fori_loop(0, CHU // 4, step, 0)
        finish_chunk(CHU - 1, (CHU - 1) % 4, (CHU - 1) % 2)
        plsc.subcore_barrier()
        for k in range(NRCH):
            r0 = s * RPS + k * RCH
            pltpu.sync_copy(acc.at[pl.ds(r0, RCH)],
                            out_hbm.at[pl.ds(c * N + r0, RCH)])

        @pl.when(s == 0)
        def _():
            pltpu.sync_copy(acc.at[pl.ds(TAIL0, TAIL)],
                            out_hbm.at[pl.ds(c * N + TAIL0, TAIL)])

    return run(src, dst, hs)


def _dinv_block(degp_ref):
    d = degp_ref[0, :, 0:1] + degp_ref[1, :, 0:1] + 1.0
    return lax.rsqrt(d)


def _row_spec():
    return pl.BlockSpec((BN, C), lambda j: (j, 0))


def _full_spec(shape):
    return pl.BlockSpec(shape, lambda j: tuple(0 for _ in shape))


def _degp_spec():
    return pl.BlockSpec((2, BN, C), lambda j: (0, j, 0))


def _smem_spec():
    return pl.BlockSpec(memory_space=pltpu.SMEM)


def _tc_embed_body(x_ref, we_ref, be_ref, wg_ref, degp_ref,
                   h_ref, hs_ref, gb_ref, nb_ref, gacc, nacc):
    j = pl.program_id(0)
    h = jnp.dot(x_ref[...], we_ref[...], precision=_HIGHEST,
                preferred_element_type=jnp.float32) + be_ref[...]
    h_ref[...] = h
    hl = jnp.dot(h, wg_ref[...], precision=_HIGHEST,
                 preferred_element_type=jnp.float32)
    hs_ref[...] = hl * _dinv_block(degp_ref)

    @pl.when(j == 0)
    def _():
        gacc[...] = jnp.zeros_like(gacc)
        nacc[...] = jnp.zeros_like(nacc)

    gacc[...] += jnp.broadcast_to(jnp.sum(h, axis=0, keepdims=True), (8, C))
    nacc[...] += lax.dot_general(h, h, (((0,), (0,)), ((), ())),
                                 precision=_HIGHEST,
                                 preferred_element_type=jnp.float32)

    @pl.when(j == NB - 1)
    def _():
        gb_ref[...] = gacc[...]
        nb_ref[...] = nacc[...]


def _tc_embed(x, W_emb, b_emb, Wg0, degp):
    return pl.pallas_call(
        _tc_embed_body,
        grid=(NB,),
        in_specs=[_row_spec(), _full_spec((C, C)), _full_spec((1, C)),
                  _full_spec((C, C)), _degp_spec()],
        out_specs=[_row_spec(), _row_spec(), _full_spec((8, C)),
                   _full_spec((C, C))],
        out_shape=[jax.ShapeDtypeStruct((N, C), jnp.float32),
                   jax.ShapeDtypeStruct((N, C), jnp.float32),
                   jax.ShapeDtypeStruct((8, C), jnp.float32),
                   jax.ShapeDtypeStruct((C, C), jnp.float32)],
        scratch_shapes=[pltpu.VMEM((8, C), jnp.float32),
                        pltpu.VMEM((C, C), jnp.float32)],
    )(x, W_emb, b_emb, Wg0, degp)


def _tc_msum_body(aggp_ref, hs_ref, degp_ref, bg_ref,
                  m_ref, s1_ref, s2_ref, a1, a2):
    j = pl.program_id(0)
    m = (aggp_ref[0] + aggp_ref[1] + hs_ref[...]) * _dinv_block(degp_ref) \
        + bg_ref[...]
    m_ref[...] = m

    @pl.when(j == 0)
    def _():
        a1[...] = jnp.zeros_like(a1)
        a2[...] = jnp.zeros_like(a2)

    a1[...] += jnp.broadcast_to(jnp.sum(m, axis=0, keepdims=True), (8, C))
    a2[...] += jnp.broadcast_to(jnp.sum(m * m, axis=0, keepdims=True), (8, C))

    @pl.when(j == NB - 1)
    def _():
        s1_ref[...] = jnp.full((8, C), jnp.sum(a1[0:1, :]), jnp.float32)
        s2_ref[...] = jnp.full((8, C), jnp.sum(a2[0:1, :]), jnp.float32)


def _tc_msum(aggp, hs, degp, bg):
    return pl.pallas_call(
        _tc_msum_body,
        grid=(NB,),
        in_specs=[pl.BlockSpec((2, BN, C), lambda j: (0, j, 0)), _row_spec(),
                  _degp_spec(), _full_spec((1, C))],
        out_specs=[_row_spec(), _full_spec((8, C)), _full_spec((8, C))],
        out_shape=[jax.ShapeDtypeStruct((N, C), jnp.float32),
                   jax.ShapeDtypeStruct((8, C), jnp.float32),
                   jax.ShapeDtypeStruct((8, C), jnp.float32)],
        scratch_shapes=[pltpu.VMEM((8, C), jnp.float32),
                        pltpu.VMEM((8, C), jnp.float32)],
    )(aggp, hs, degp, bg)


_INV_NC = 1.0 / (N * C)
_EPS = 1e-5


def _ln_scale(s1_ref, s2_ref):
    mean = s1_ref[0:1, 0:1] * _INV_NC
    var = s2_ref[0:1, 0:1] * _INV_NC - mean * mean
    rstd = lax.rsqrt(var + _EPS)
    return mean, rstd


def _tc_ffn_body(m_ref, s1_ref, s2_ref, hp_ref, wf_ref, bf_ref,
                 lnw_ref, lnb_ref, f_ref, t1_ref, t2_ref, a1, a2):
    j = pl.program_id(0)
    mean, rstd = _ln_scale(s1_ref, s2_ref)
    mhat = (m_ref[...] - mean) * rstd * lnw_ref[0, 0] + lnb_ref[0, 0]
    h = hp_ref[...] + jnp.maximum(mhat, 0.0)
    f = jnp.dot(h, wf_ref[...], precision=_HIGHEST,
                preferred_element_type=jnp.float32) + bf_ref[...]
    f_ref[...] = f

    @pl.when(j == 0)
    def _():
        a1[...] = jnp.zeros_like(a1)
        a2[...] = jnp.zeros_like(a2)

    a1[...] += jnp.broadcast_to(jnp.sum(f, axis=0, keepdims=True), (8, C))
    a2[...] += jnp.broadcast_to(jnp.sum(f * f, axis=0, keepdims=True), (8, C))

    @pl.when(j == NB - 1)
    def _():
        t1_ref[...] = jnp.full((8, C), jnp.sum(a1[0:1, :]), jnp.float32)
        t2_ref[...] = jnp.full((8, C), jnp.sum(a2[0:1, :]), jnp.float32)


def _tc_ffn(m, s1, s2, hprev, Wf_i, bf_i, lnw, lnb):
    return pl.pallas_call(
        _tc_ffn_body,
        grid=(NB,),
        in_specs=[_row_spec(), _full_spec((8, C)), _full_spec((8, C)),
                  _row_spec(), _full_spec((C, C)), _full_spec((1, C)),
                  _smem_spec(), _smem_spec()],
        out_specs=[_row_spec(), _full_spec((8, C)), _full_spec((8, C))],
        out_shape=[jax.ShapeDtypeStruct((N, C), jnp.float32),
                   jax.ShapeDtypeStruct((8, C), jnp.float32),
                   jax.ShapeDtypeStruct((8, C), jnp.float32)],
        scratch_shapes=[pltpu.VMEM((8, C), jnp.float32),
                        pltpu.VMEM((8, C), jnp.float32)],
    )(m, s1, s2, hprev, Wf_i, bf_i, lnw, lnb)


def _tc_out_body(f_ref, s1_ref, s2_ref, lnw_ref, lnb_ref, degp_ref, wg_ref,
                 h_ref, hs_ref, gb_ref, nb_ref, gacc, nacc):
    j = pl.program_id(0)
    mean, rstd = _ln_scale(s1_ref, s2_ref)
    fhat = (f_ref[...] - mean) * rstd * lnw_ref[0, 0] + lnb_ref[0, 0]
    h = jnp.maximum(fhat, 0.0)
    h_ref[...] = h
    hl = jnp.dot(h, wg_ref[...], precision=_HIGHEST,
                 preferred_element_type=jnp.float32)
    hs_ref[...] = hl * _dinv_block(degp_ref)

    @pl.when(j == 0)
    def _():
        gacc[...] = jnp.zeros_like(gacc)
        nacc[...] = jnp.zeros_like(nacc)

    gacc[...] += jnp.broadcast_to(jnp.sum(h, axis=0, keepdims=True), (8, C))
    nacc[...] += lax.dot_general(h, h, (((0,), (0,)), ((), ())),
                                 precision=_HIGHEST,
                                 preferred_element_type=jnp.float32)

    @pl.when(j == NB - 1)
    def _():
        gb_ref[...] = gacc[...]
        nb_ref[...] = nacc[...]


def _tc_out(f, s1, s2, lnw, lnb, degp, Wg_next):
    return pl.pallas_call(
        _tc_out_body,
        grid=(NB,),
        in_specs=[_row_spec(), _full_spec((8, C)), _full_spec((8, C)),
                  _smem_spec(), _smem_spec(), _degp_spec(),
                  _full_spec((C, C))],
        out_specs=[_row_spec(), _row_spec(), _full_spec((8, C)),
                   _full_spec((C, C))],
        out_shape=[jax.ShapeDtypeStruct((N, C), jnp.float32),
                   jax.ShapeDtypeStruct((N, C), jnp.float32),
                   jax.ShapeDtypeStruct((8, C), jnp.float32),
                   jax.ShapeDtypeStruct((C, C), jnp.float32)],
        scratch_shapes=[pltpu.VMEM((8, C), jnp.float32),
                        pltpu.VMEM((C, C), jnp.float32)],
    )(f, s1, s2, lnw, lnb, degp, Wg_next)


def _tc_out_last_body(f_ref, s1_ref, s2_ref, lnw_ref, lnb_ref,
                      gb_ref, nb_ref, gacc, nacc):
    j = pl.program_id(0)
    mean, rstd = _ln_scale(s1_ref, s2_ref)
    fhat = (f_ref[...] - mean) * rstd * lnw_ref[0, 0] + lnb_ref[0, 0]
    h = jnp.maximum(fhat, 0.0)

    @pl.when(j == 0)
    def _():
        gacc[...] = jnp.zeros_like(gacc)
        nacc[...] = jnp.zeros_like(nacc)

    gacc[...] += jnp.broadcast_to(jnp.sum(h, axis=0, keepdims=True), (8, C))
    nacc[...] += lax.dot_general(h, h, (((0,), (0,)), ((), ())),
                                 precision=_HIGHEST,
                                 preferred_element_type=jnp.float32)

    @pl.when(j == NB - 1)
    def _():
        gb_ref[...] = gacc[...]
        nb_ref[...] = nacc[...]


def _tc_out_last(f, s1, s2, lnw, lnb):
    return pl.pallas_call(
        _tc_out_last_body,
        grid=(NB,),
        in_specs=[_row_spec(), _full_spec((8, C)), _full_spec((8, C)),
                  _smem_spec(), _smem_spec()],
        out_specs=[_full_spec((8, C)), _full_spec((C, C))],
        out_shape=[jax.ShapeDtypeStruct((8, C), jnp.float32),
                   jax.ShapeDtypeStruct((C, C), jnp.float32)],
        scratch_shapes=[pltpu.VMEM((8, C), jnp.float32),
                        pltpu.VMEM((C, C), jnp.float32)],
    )(f, s1, s2, lnw, lnb)


def kernel(x, edge_index, batch, W_emb, b_emb, Wg, bg, ln1w, ln1b,
           Wf, bf, ln2w, ln2b):
    src = edge_index[0]
    dst = edge_index[1]
    degp = _sc_deg(dst).reshape(2, N, C)

    h0, hs0, gb0, nb0 = _tc_embed(x, W_emb, b_emb[None, :], Wg[0], degp)

    gbs, nbs = [gb0], [nb0]
    h, hs = h0, hs0
    for i in range(L):
        aggp = _sc_agg(src, dst, hs).reshape(2, N, C)
        m, s1, s2 = _tc_msum(aggp, hs, degp, bg[i][None, :])
        f, t1, t2 = _tc_ffn(m, s1, s2, h, Wf[i], bf[i][None, :],
                            ln1w[i].reshape(1, 1), ln1b[i].reshape(1, 1))
        if i + 1 < L:
            h, hs, gb_i, nb_i = _tc_out(f, t1, t2, ln2w[i].reshape(1, 1),
                                        ln2b[i].reshape(1, 1), degp, Wg[i + 1])
        else:
            gb_i, nb_i = _tc_out_last(f, t1, t2, ln2w[i].reshape(1, 1),
                                      ln2b[i].reshape(1, 1))
        gbs.append(gb_i)
        nbs.append(nb_i)

    gb_out = jnp.concatenate([g[0:1] for g in gbs], axis=-1)
    nb_out = jnp.stack(nbs, axis=0)[None]
    return gb_out, nb_out


# speedup vs baseline: 22.4843x; 1.0342x over previous
import functools

import jax
import jax.numpy as jnp
from jax import lax
from jax.experimental import pallas as pl
from jax.experimental.pallas import tpu as pltpu
from jax.experimental.pallas import tpu_sc as plsc

N = 10000
E = 320000
C = 128
L = 2

BN = 1000
NB = N // BN

NSC = 2
NSUB = 16
NW = NSC * NSUB
K = 128
NCH = E // K
CH_PER_W = -(-NCH // NW)
RPS = 624
RCH = 104
NRCH = RPS // RCH
TAIL0 = NSUB * RPS
TAIL = N - TAIL0

_HIGHEST = jax.lax.Precision.HIGHEST


def _sc_mesh():
    return plsc.VectorSubcoreMesh(core_axis_name="c", subcore_axis_name="s")


def _sc_deg(dst):
    @functools.partial(
        pl.kernel,
        out_type=jax.ShapeDtypeStruct((2 * N, C), jnp.float32),
        mesh=_sc_mesh(),
        scratch_types=[
            pltpu.VMEM((K,), jnp.int32),
            pltpu.VMEM((K, C), jnp.float32),
            pltpu.VMEM((RCH, C), jnp.float32),
            pltpu.VMEM_SHARED((N, C), jnp.float32),
        ],
    )
    def run(dst_hbm, out_hbm, idx_v, ones_v, zb, acc):
        c = lax.axis_index("c")
        s = lax.axis_index("s")
        wid = c * NSUB + s

        def fill_ones(i, carry):
            for j in range(C // 16):
                ones_v[i, pl.ds(j * 16, 16)] = jnp.full((16,), 1.0,
                                                        jnp.float32)
            return carry

        lax.fori_loop(0, K, fill_ones, 0)

        def fill_zero(i, carry):
            for j in range(C // 16):
                zb[i, pl.ds(j * 16, 16)] = jnp.zeros((16,), jnp.float32)
            return carry

        lax.fori_loop(0, RCH, fill_zero, 0)
        for k in range(NRCH):
            pltpu.sync_copy(zb, acc.at[pl.ds(s * RPS + k * RCH, RCH)])

        @pl.when(s == 0)
        def _():
            pltpu.sync_copy(zb.at[pl.ds(0, TAIL)], acc.at[pl.ds(TAIL0, TAIL)])

        plsc.subcore_barrier()

        def chunk(i, carry):
            cid = wid + i * NW

            @pl.when(cid < NCH)
            def _():
                pltpu.sync_copy(dst_hbm.at[pl.ds(cid * K, K)], idx_v)
                pltpu.sync_copy(ones_v, acc.at[idx_v], add=True)

            return carry

        lax.fori_loop(0, CH_PER_W, chunk, 0)
        plsc.subcore_barrier()
        for k in range(NRCH):
            r0 = s * RPS + k * RCH
            pltpu.sync_copy(acc.at[pl.ds(r0, RCH)],
                            out_hbm.at[pl.ds(c * N + r0, RCH)])

        @pl.when(s == 0)
        def _():
            pltpu.sync_copy(acc.at[pl.ds(TAIL0, TAIL)],
                            out_hbm.at[pl.ds(c * N + TAIL0, TAIL)])

    return run(dst)


CHU = 80


def _sc_agg(src, dst, hs):
    @functools.partial(
        pl.kernel,
        out_type=jax.ShapeDtypeStruct((2 * N, C), jnp.float32),
        mesh=_sc_mesh(),
        scratch_types=(
            [pltpu.VMEM((K,), jnp.int32) for _ in range(8)]
            + [pltpu.VMEM((K, C), jnp.float32) for _ in range(2)]
            + [pltpu.VMEM((RCH, C), jnp.float32),
               pltpu.VMEM_SHARED((N, C), jnp.float32)]
            + [pltpu.SemaphoreType.DMA for _ in range(10)]
        ),
    )
    def run(src_hbm, dst_hbm, hs_hbm, out_hbm,
            is0, is1, is2, is3, id0, id1, id2, id3, rows0, rows1, zb, acc,
            gis0, gis1, gis2, gis3, gid0, gid1, gid2, gid3, gg0, gg1):
        isl = [is0, is1, is2, is3]
        idl = [id0, id1, id2, id3]
        rows = [rows0, rows1]
        sis = [gis0, gis1, gis2, gis3]
        sid = [gid0, gid1, gid2, gid3]
        sg = [gg0, gg1]
        c = lax.axis_index("c")
        s = lax.axis_index("s")
        wid = c * NSUB + s

        def fill_zero(i, carry):
            for j in range(C // 16):
                zb[i, pl.ds(j * 16, 16)] = jnp.zeros((16,), jnp.float32)
            return carry

        lax.fori_loop(0, RCH, fill_zero, 0)
        for k in range(NRCH):
            pltpu.sync_copy(zb, acc.at[pl.ds(s * RPS + k * RCH, RCH)])

        @pl.when(s == 0)
        def _():
            pltpu.sync_copy(zb.at[pl.ds(0, TAIL)], acc.at[pl.ds(TAIL0, TAIL)])

        plsc.subcore_barrier()

        def base_of(i):
            cid = wid + i * NW
            return jnp.where(cid < NCH, cid * K, 0), cid < NCH

        def issue_idx(i, slot):
            base, _ = base_of(i)
            pltpu.make_async_copy(src_hbm.at[pl.ds(base, K)], isl[slot],
                                  sis[slot]).start()
            pltpu.make_async_copy(dst_hbm.at[pl.ds(base, K)], idl[slot],
                                  sid[slot]).start()

        def wait_idx_s(slot):
            pltpu.make_async_copy(src_hbm.at[pl.ds(0, K)], isl[slot],
                                  sis[slot]).wait()

        def finish_chunk(i, slot, rb):
            pltpu.make_async_copy(hs_hbm.at[isl[slot]], rows[rb],
                                  sg[rb]).wait()
            pltpu.make_async_copy(dst_hbm.at[pl.ds(0, K)], idl[slot],
                                  sid[slot]).wait()
            _, valid = base_of(i)

            @pl.when(valid)
            def _():
                pltpu.sync_copy(rows[rb], acc.at[idl[slot]], add=True)

        for u in range(3):
            issue_idx(u, u)

        def step(p, carry):
            for u in range(4):
                i = 4 * p + u
                wait_idx_s(u)
                pltpu.make_async_copy(hs_hbm.at[isl[u]], rows[u % 2],
                                      sg[u % 2]).start()
                @pl.when(i >= 1)
                def _(u=u, i=i):
                    finish_chunk(i - 1, (u - 1) % 4, (u - 1) % 2)

                @pl.when(i + 3 < CHU)
                def _(u=u, i=i):
                    issue_idx(i + 3, (u + 3) % 4)

            return carry

        lax.fori_loop(0, CHU // 4, step, 0)
        finish_chunk(CHU - 1, (CHU - 1) % 4, (CHU - 1) % 2)
        plsc.subcore_barrier()
        for k in range(NRCH):
            r0 = s * RPS + k * RCH
            pltpu.sync_copy(acc.at[pl.ds(r0, RCH)],
                            out_hbm.at[pl.ds(c * N + r0, RCH)])

        @pl.when(s == 0)
        def _():
            pltpu.sync_copy(acc.at[pl.ds(TAIL0, TAIL)],
                            out_hbm.at[pl.ds(c * N + TAIL0, TAIL)])

    return run(src, dst, hs)


def _dinv_block(degp_ref):
    d = degp_ref[0, :, 0:1] + degp_ref[1, :, 0:1] + 1.0
    return lax.rsqrt(d)


def _row_spec():
    return pl.BlockSpec((BN, C), lambda j: (j, 0))


def _full_spec(shape):
    return pl.BlockSpec(shape, lambda j: tuple(0 for _ in shape))


def _degp_spec():
    return pl.BlockSpec((2, BN, C), lambda j: (0, j, 0))


def _smem_spec():
    return pl.BlockSpec(memory_space=pltpu.SMEM)


def _tc_embed_body(x_ref, we_ref, be_ref, wg_ref, degp_ref,
                   h_ref, hs_ref, dinv8_ref, gb_ref, nb_ref, gacc, nacc):
    j = pl.program_id(0)
    h = jnp.dot(x_ref[...], we_ref[...], precision=_HIGHEST,
                preferred_element_type=jnp.float32) + be_ref[...]
    h_ref[...] = h
    hl = jnp.dot(h, wg_ref[...], precision=_HIGHEST,
                 preferred_element_type=jnp.float32)
    dinv = _dinv_block(degp_ref)
    dinv8_ref[...] = jnp.broadcast_to(dinv, (BN, 8))
    hs_ref[...] = hl * dinv

    @pl.when(j == 0)
    def _():
        gacc[...] = jnp.zeros_like(gacc)
        nacc[...] = jnp.zeros_like(nacc)

    gacc[...] += jnp.broadcast_to(jnp.sum(h, axis=0, keepdims=True), (8, C))
    nacc[...] += lax.dot_general(h, h, (((0,), (0,)), ((), ())),
                                 precision=_HIGHEST,
                                 preferred_element_type=jnp.float32)

    @pl.when(j == NB - 1)
    def _():
        gb_ref[...] = gacc[...]
        nb_ref[...] = nacc[...]


def _tc_embed(x, W_emb, b_emb, Wg0, degp):
    return pl.pallas_call(
        _tc_embed_body,
        grid=(NB,),
        in_specs=[_row_spec(), _full_spec((C, C)), _full_spec((1, C)),
                  _full_spec((C, C)), _degp_spec()],
        out_specs=[_row_spec(), _row_spec(),
                   pl.BlockSpec((BN, 8), lambda j: (j, 0)),
                   _full_spec((8, C)), _full_spec((C, C))],
        out_shape=[jax.ShapeDtypeStruct((N, C), jnp.float32),
                   jax.ShapeDtypeStruct((N, C), jnp.float32),
                   jax.ShapeDtypeStruct((N, 8), jnp.float32),
                   jax.ShapeDtypeStruct((8, C), jnp.float32),
                   jax.ShapeDtypeStruct((C, C), jnp.float32)],
        scratch_shapes=[pltpu.VMEM((8, C), jnp.float32),
                        pltpu.VMEM((C, C), jnp.float32)],
    )(x, W_emb, b_emb, Wg0, degp)


_INV_NC = 1.0 / (N * C)
_EPS = 1e-5


def _dinv8_spec_ph(phases):
    def imap(p, j):
        use = (p == phases[0])
        for q in phases[1:]:
            use = use | (p == q)
        return (jnp.where(use, j, 0), 0)
    return pl.BlockSpec((BN, 8), imap)


def _row_spec_ph(phase):
    return pl.BlockSpec((BN, C), lambda p, j: (jnp.where(p == phase, j, 0), 0))


def _full_spec2(shape):
    return pl.BlockSpec(shape, lambda p, j: tuple(0 for _ in shape))


def _smem_spec2():
    return pl.BlockSpec(memory_space=pltpu.SMEM)


def _make_tc_layer_body(has_next):
    def body(*refs):
        if has_next:
            (aggp_ref, hs_ref, dinv8_ref, bg_ref, hp_ref, wf_ref, bf_ref,
             ln1w_ref, ln1b_ref, ln2w_ref, ln2b_ref, wgn_ref,
             hn_ref, hsn_ref, gb_ref, nb_ref,
             m_all, f_all, sm1, sm2, sf1, sf2, gacc, nacc) = refs
        else:
            (aggp_ref, hs_ref, dinv8_ref, bg_ref, hp_ref, wf_ref, bf_ref,
             ln1w_ref, ln1b_ref, ln2w_ref, ln2b_ref,
             gb_ref, nb_ref,
             m_all, f_all, sm1, sm2, sf1, sf2, gacc, nacc) = refs
        p = pl.program_id(0)
        j = pl.program_id(1)
        rows = pl.ds(j * BN, BN)

        @pl.when(p == 0)
        def _():
            dinv = dinv8_ref[:, 0:1]
            m = (aggp_ref[0] + aggp_ref[1] + hs_ref[...]) * dinv + bg_ref[...]
            m_all[rows, :] = m

            @pl.when(j == 0)
            def _():
                sm1[0, 0] = 0.0
                sm2[0, 0] = 0.0

            sm1[0, 0] += jnp.sum(m)
            sm2[0, 0] += jnp.sum(m * m)

        @pl.when(p == 1)
        def _():
            mean = sm1[0, 0] * _INV_NC
            var = sm2[0, 0] * _INV_NC - mean * mean
            rstd = lax.rsqrt(var + _EPS)
            mhat = ((m_all[rows, :] - mean) * rstd * ln1w_ref[0, 0]
                    + ln1b_ref[0, 0])
            hmid = hp_ref[...] + jnp.maximum(mhat, 0.0)
            f = jnp.dot(hmid, wf_ref[...], precision=_HIGHEST,
                        preferred_element_type=jnp.float32) + bf_ref[...]
            f_all[rows, :] = f

            @pl.when(j == 0)
            def _():
                sf1[0, 0] = 0.0
                sf2[0, 0] = 0.0

            sf1[0, 0] += jnp.sum(f)
            sf2[0, 0] += jnp.sum(f * f)

        @pl.when(p == 2)
        def _():
            mean = sf1[0, 0] * _INV_NC
            var = sf2[0, 0] * _INV_NC - mean * mean
            rstd = lax.rsqrt(var + _EPS)
            fhat = ((f_all[rows, :] - mean) * rstd * ln2w_ref[0, 0]
                    + ln2b_ref[0, 0])
            hn = jnp.maximum(fhat, 0.0)

            @pl.when(j == 0)
            def _():
                gacc[...] = jnp.zeros_like(gacc)
                nacc[...] = jnp.zeros_like(nacc)

            gacc[...] += jnp.broadcast_to(
                jnp.sum(hn, axis=0, keepdims=True), (8, C))
            nacc[...] += lax.dot_general(hn, hn, (((0,), (0,)), ((), ())),
                                         precision=_HIGHEST,
                                         preferred_element_type=jnp.float32)
            if has_next:
                hn_ref[...] = hn
                hl = jnp.dot(hn, wgn_ref[...], precision=_HIGHEST,
                             preferred_element_type=jnp.float32)
                hsn_ref[...] = hl * dinv8_ref[:, 0:1]

            @pl.when(j == NB - 1)
            def _():
                gb_ref[...] = gacc[...]
                nb_ref[...] = nacc[...]

    return body


def _tc_layer(aggp, hs, dinv8, bg_i, hprev, Wf_i, bf_i, ln1w, ln1b,
              ln2w, ln2b, Wg_next):
    has_next = Wg_next is not None
    in_specs = [
        pl.BlockSpec((2, BN, C), lambda p, j: (0, jnp.where(p == 0, j, 0), 0)),
        _row_spec_ph(0),
        _dinv8_spec_ph((0, 2) if has_next else (0,)),
        _full_spec2((1, C)),
        _row_spec_ph(1),
        _full_spec2((C, C)),
        _full_spec2((1, C)),
        _smem_spec2(), _smem_spec2(), _smem_spec2(), _smem_spec2(),
    ]
    args = [aggp, hs, dinv8, bg_i, hprev, Wf_i, bf_i, ln1w, ln1b, ln2w, ln2b]
    if has_next:
        in_specs.append(_full_spec2((C, C)))
        args.append(Wg_next)
        out_specs = [_row_spec_ph(2), _row_spec_ph(2),
                     _full_spec2((8, C)), _full_spec2((C, C))]
        out_shape = [jax.ShapeDtypeStruct((N, C), jnp.float32),
                     jax.ShapeDtypeStruct((N, C), jnp.float32),
                     jax.ShapeDtypeStruct((8, C), jnp.float32),
                     jax.ShapeDtypeStruct((C, C), jnp.float32)]
    else:
        out_specs = [_full_spec2((8, C)), _full_spec2((C, C))]
        out_shape = [jax.ShapeDtypeStruct((8, C), jnp.float32),
                     jax.ShapeDtypeStruct((C, C), jnp.float32)]
    return pl.pallas_call(
        _make_tc_layer_body(has_next),
        grid=(3, NB),
        in_specs=in_specs,
        out_specs=out_specs,
        out_shape=out_shape,
        scratch_shapes=[pltpu.VMEM((N, C), jnp.float32),
                        pltpu.VMEM((N, C), jnp.float32),
                        pltpu.SMEM((1, 1), jnp.float32),
                        pltpu.SMEM((1, 1), jnp.float32),
                        pltpu.SMEM((1, 1), jnp.float32),
                        pltpu.SMEM((1, 1), jnp.float32),
                        pltpu.VMEM((8, C), jnp.float32),
                        pltpu.VMEM((C, C), jnp.float32)],
    )(*args)


def kernel(x, edge_index, batch, W_emb, b_emb, Wg, bg, ln1w, ln1b,
           Wf, bf, ln2w, ln2b):
    src = edge_index[0]
    dst = edge_index[1]
    degp = _sc_deg(dst).reshape(2, N, C)

    h0, hs0, dinv8, gb0, nb0 = _tc_embed(x, W_emb, b_emb[None, :], Wg[0],
                                         degp)

    gbs, nbs = [gb0], [nb0]
    h, hs = h0, hs0
    for i in range(L):
        aggp = _sc_agg(src, dst, hs).reshape(2, N, C)
        outs = _tc_layer(aggp, hs, dinv8, bg[i][None, :], h,
                         Wf[i], bf[i][None, :],
                         ln1w[i].reshape(1, 1), ln1b[i].reshape(1, 1),
                         ln2w[i].reshape(1, 1), ln2b[i].reshape(1, 1),
                         Wg[i + 1] if i + 1 < L else None)
        if i + 1 < L:
            h, hs, gb_i, nb_i = outs
        else:
            gb_i, nb_i = outs
        gbs.append(gb_i)
        nbs.append(nb_i)

    gb_out = jnp.concatenate([g[0:1] for g in gbs], axis=-1)
    nb_out = jnp.stack(nbs, axis=0)[None]
    return gb_out, nb_out


# speedup vs baseline: 24.8483x; 1.1051x over previous
import functools

import jax
import jax.numpy as jnp
from jax import lax
from jax.experimental import pallas as pl
from jax.experimental.pallas import tpu as pltpu
from jax.experimental.pallas import tpu_sc as plsc

N = 10000
E = 320000
C = 128
L = 2

BN = 1000
NB = N // BN

NSC = 2
NSUB = 16
NW = NSC * NSUB
K = 128
NCH = E // K
CH_PER_W = -(-NCH // NW)
RPS = 624
RCH = 104
NRCH = RPS // RCH
TAIL0 = NSUB * RPS
TAIL = N - TAIL0

_HIGHEST = jax.lax.Precision.HIGHEST


def _sc_mesh():
    return plsc.VectorSubcoreMesh(core_axis_name="c", subcore_axis_name="s")


def _sc_deg(dst):
    @functools.partial(
        pl.kernel,
        out_type=jax.ShapeDtypeStruct((2 * N,), jnp.float32),
        mesh=_sc_mesh(),
        scratch_types=[
            pltpu.VMEM((K,), jnp.int32),
            pltpu.VMEM((K,), jnp.float32),
            pltpu.VMEM((RPS,), jnp.float32),
            pltpu.VMEM_SHARED((N,), jnp.float32),
        ],
    )
    def run(dst_hbm, out_hbm, idx_v, ones_v, zb, acc):
        c = lax.axis_index("c")
        s = lax.axis_index("s")
        wid = c * NSUB + s

        def fill_ones(i, carry):
            ones_v[pl.ds(i * 16, 16)] = jnp.full((16,), 1.0, jnp.float32)
            return carry

        lax.fori_loop(0, K // 16, fill_ones, 0)

        def fill_zero(i, carry):
            zb[pl.ds(i * 16, 16)] = jnp.zeros((16,), jnp.float32)
            return carry

        lax.fori_loop(0, RPS // 16, fill_zero, 0)
        pltpu.sync_copy(zb, acc.at[pl.ds(s * RPS, RPS)])

        @pl.when(s == 0)
        def _():
            pltpu.sync_copy(zb.at[pl.ds(0, TAIL)], acc.at[pl.ds(TAIL0, TAIL)])

        plsc.subcore_barrier()

        def chunk(i, carry):
            cid = wid + i * NW

            @pl.when(cid < NCH)
            def _():
                pltpu.sync_copy(dst_hbm.at[pl.ds(cid * K, K)], idx_v)
                pltpu.sync_copy(ones_v, acc.at[idx_v], add=True)

            return carry

        lax.fori_loop(0, CH_PER_W, chunk, 0)
        plsc.subcore_barrier()
        pltpu.sync_copy(acc.at[pl.ds(s * RPS, RPS)], zb)
        pltpu.sync_copy(zb, out_hbm.at[pl.ds(c * N + s * RPS, RPS)])

        @pl.when(s == 0)
        def _():
            pltpu.sync_copy(acc.at[pl.ds(TAIL0, TAIL)], zb.at[pl.ds(0, TAIL)])
            pltpu.sync_copy(zb.at[pl.ds(0, TAIL)],
                            out_hbm.at[pl.ds(c * N + TAIL0, TAIL)])

    return run(dst)


CHU = 80


def _sc_agg(src, dst, hs):
    @functools.partial(
        pl.kernel,
        out_type=jax.ShapeDtypeStruct((2 * N, C), jnp.float32),
        mesh=_sc_mesh(),
        scratch_types=(
            [pltpu.VMEM((K,), jnp.int32) for _ in range(8)]
            + [pltpu.VMEM((K, C), jnp.float32) for _ in range(2)]
            + [pltpu.VMEM((RCH, C), jnp.float32),
               pltpu.VMEM_SHARED((N, C), jnp.float32)]
            + [pltpu.SemaphoreType.DMA for _ in range(10)]
        ),
    )
    def run(src_hbm, dst_hbm, hs_hbm, out_hbm,
            is0, is1, is2, is3, id0, id1, id2, id3, rows0, rows1, zb, acc,
            gis0, gis1, gis2, gis3, gid0, gid1, gid2, gid3, gg0, gg1):
        isl = [is0, is1, is2, is3]
        idl = [id0, id1, id2, id3]
        rows = [rows0, rows1]
        sis = [gis0, gis1, gis2, gis3]
        sid = [gid0, gid1, gid2, gid3]
        sg = [gg0, gg1]
        c = lax.axis_index("c")
        s = lax.axis_index("s")
        wid = c * NSUB + s

        def fill_zero(i, carry):
            for j in range(C // 16):
                zb[i, pl.ds(j * 16, 16)] = jnp.zeros((16,), jnp.float32)
            return carry

        lax.fori_loop(0, RCH, fill_zero, 0)
        for k in range(NRCH):
            pltpu.sync_copy(zb, acc.at[pl.ds(s * RPS + k * RCH, RCH)])

        @pl.when(s == 0)
        def _():
            pltpu.sync_copy(zb.at[pl.ds(0, TAIL)], acc.at[pl.ds(TAIL0, TAIL)])

        plsc.subcore_barrier()

        def base_of(i):
            cid = wid + i * NW
            return jnp.where(cid < NCH, cid * K, 0), cid < NCH

        def issue_idx(i, slot):
            base, _ = base_of(i)
            pltpu.make_async_copy(src_hbm.at[pl.ds(base, K)], isl[slot],
                                  sis[slot]).start()
            pltpu.make_async_copy(dst_hbm.at[pl.ds(base, K)], idl[slot],
                                  sid[slot]).start()

        def wait_idx_s(slot):
            pltpu.make_async_copy(src_hbm.at[pl.ds(0, K)], isl[slot],
                                  sis[slot]).wait()

        def finish_chunk(i, slot, rb):
            pltpu.make_async_copy(hs_hbm.at[isl[slot]], rows[rb],
                                  sg[rb]).wait()
            pltpu.make_async_copy(dst_hbm.at[pl.ds(0, K)], idl[slot],
                                  sid[slot]).wait()
            _, valid = base_of(i)

            @pl.when(valid)
            def _():
                pltpu.sync_copy(rows[rb], acc.at[idl[slot]], add=True)

        for u in range(3):
            issue_idx(u, u)

        def step(p, carry):
            for u in range(4):
                i = 4 * p + u
                wait_idx_s(u)
                pltpu.make_async_copy(hs_hbm.at[isl[u]], rows[u % 2],
                                      sg[u % 2]).start()
                @pl.when(i >= 1)
                def _(u=u, i=i):
                    finish_chunk(i - 1, (u - 1) % 4, (u - 1) % 2)

                @pl.when(i + 3 < CHU)
                def _(u=u, i=i):
                    issue_idx(i + 3, (u + 3) % 4)

            return carry

        lax.fori_loop(0, CHU // 4, step, 0)
        finish_chunk(CHU - 1, (CHU - 1) % 4, (CHU - 1) % 2)
        plsc.subcore_barrier()
        for k in range(NRCH):
            r0 = s * RPS + k * RCH
            pltpu.sync_copy(acc.at[pl.ds(r0, RCH)],
                            out_hbm.at[pl.ds(c * N + r0, RCH)])

        @pl.when(s == 0)
        def _():
            pltpu.sync_copy(acc.at[pl.ds(TAIL0, TAIL)],
                            out_hbm.at[pl.ds(c * N + TAIL0, TAIL)])

    return run(src, dst, hs)


def _dinv_block(degp_ref):
    d = degp_ref[0, :, 0:1] + degp_ref[1, :, 0:1] + 1.0
    return lax.rsqrt(d)


def _row_spec():
    return pl.BlockSpec((BN, C), lambda j: (j, 0))


def _full_spec(shape):
    return pl.BlockSpec(shape, lambda j: tuple(0 for _ in shape))


def _degp_spec():
    return pl.BlockSpec((2, BN, 8), lambda j: (0, j, 0))


def _smem_spec():
    return pl.BlockSpec(memory_space=pltpu.SMEM)


def _tc_embed_body(x_ref, we_ref, be_ref, wg_ref, degp_ref,
                   h_ref, hs_ref, dinv8_ref, gb_ref, nb_ref, gacc, nacc):
    j = pl.program_id(0)
    h = jnp.dot(x_ref[...], we_ref[...], precision=_HIGHEST,
                preferred_element_type=jnp.float32) + be_ref[...]
    h_ref[...] = h
    hl = jnp.dot(h, wg_ref[...], precision=_HIGHEST,
                 preferred_element_type=jnp.float32)
    dinv = _dinv_block(degp_ref)
    dinv8_ref[...] = jnp.broadcast_to(dinv, (BN, 8))
    hs_ref[...] = hl * dinv

    @pl.when(j == 0)
    def _():
        gacc[...] = jnp.zeros_like(gacc)
        nacc[...] = jnp.zeros_like(nacc)

    gacc[...] += jnp.broadcast_to(jnp.sum(h, axis=0, keepdims=True), (8, C))
    nacc[...] += lax.dot_general(h, h, (((0,), (0,)), ((), ())),
                                 precision=_HIGHEST,
                                 preferred_element_type=jnp.float32)

    @pl.when(j == NB - 1)
    def _():
        gb_ref[...] = gacc[...]
        nb_ref[...] = nacc[...]


def _tc_embed(x, W_emb, b_emb, Wg0, degp):
    return pl.pallas_call(
        _tc_embed_body,
        grid=(NB,),
        in_specs=[_row_spec(), _full_spec((C, C)), _full_spec((1, C)),
                  _full_spec((C, C)), _degp_spec()],
        out_specs=[_row_spec(), _row_spec(),
                   pl.BlockSpec((BN, 8), lambda j: (j, 0)),
                   _full_spec((8, C)), _full_spec((C, C))],
        out_shape=[jax.ShapeDtypeStruct((N, C), jnp.float32),
                   jax.ShapeDtypeStruct((N, C), jnp.float32),
                   jax.ShapeDtypeStruct((N, 8), jnp.float32),
                   jax.ShapeDtypeStruct((8, C), jnp.float32),
                   jax.ShapeDtypeStruct((C, C), jnp.float32)],
        scratch_shapes=[pltpu.VMEM((8, C), jnp.float32),
                        pltpu.VMEM((C, C), jnp.float32)],
    )(x, W_emb, b_emb, Wg0, degp)


_INV_NC = 1.0 / (N * C)
_EPS = 1e-5


def _dinv8_spec_ph(phases):
    def imap(p, j):
        use = (p == phases[0])
        for q in phases[1:]:
            use = use | (p == q)
        return (jnp.where(use, j, 0), 0)
    return pl.BlockSpec((BN, 8), imap)


def _row_spec_ph(phase):
    return pl.BlockSpec((BN, C), lambda p, j: (jnp.where(p == phase, j, 0), 0))


def _full_spec2(shape):
    return pl.BlockSpec(shape, lambda p, j: tuple(0 for _ in shape))


def _smem_spec2():
    return pl.BlockSpec(memory_space=pltpu.SMEM)


def _make_tc_layer_body(has_next):
    def body(*refs):
        if has_next:
            (aggp_ref, hs_ref, dinv8_ref, bg_ref, hp_ref, wf_ref, bf_ref,
             ln1w_ref, ln1b_ref, ln2w_ref, ln2b_ref, wgn_ref,
             hn_ref, hsn_ref, gb_ref, nb_ref,
             m_all, f_all, sm1, sm2, sf1, sf2, gacc, nacc) = refs
        else:
            (aggp_ref, hs_ref, dinv8_ref, bg_ref, hp_ref, wf_ref, bf_ref,
             ln1w_ref, ln1b_ref, ln2w_ref, ln2b_ref,
             gb_ref, nb_ref,
             m_all, f_all, sm1, sm2, sf1, sf2, gacc, nacc) = refs
        p = pl.program_id(0)
        j = pl.program_id(1)
        rows = pl.ds(j * BN, BN)

        @pl.when(p == 0)
        def _():
            dinv = dinv8_ref[:, 0:1]
            m = (aggp_ref[0] + aggp_ref[1] + hs_ref[...]) * dinv + bg_ref[...]
            m_all[rows, :] = m

            @pl.when(j == 0)
            def _():
                sm1[0, 0] = 0.0
                sm2[0, 0] = 0.0

            sm1[0, 0] += jnp.sum(m)
            sm2[0, 0] += jnp.sum(m * m)

        @pl.when(p == 1)
        def _():
            mean = sm1[0, 0] * _INV_NC
            var = sm2[0, 0] * _INV_NC - mean * mean
            rstd = lax.rsqrt(var + _EPS)
            mhat = ((m_all[rows, :] - mean) * rstd * ln1w_ref[0, 0]
                    + ln1b_ref[0, 0])
            hmid = hp_ref[...] + jnp.maximum(mhat, 0.0)
            f = jnp.dot(hmid, wf_ref[...], precision=_HIGHEST,
                        preferred_element_type=jnp.float32) + bf_ref[...]
            f_all[rows, :] = f

            @pl.when(j == 0)
            def _():
                sf1[0, 0] = 0.0
                sf2[0, 0] = 0.0

            sf1[0, 0] += jnp.sum(f)
            sf2[0, 0] += jnp.sum(f * f)

        @pl.when(p == 2)
        def _():
            mean = sf1[0, 0] * _INV_NC
            var = sf2[0, 0] * _INV_NC - mean * mean
            rstd = lax.rsqrt(var + _EPS)
            fhat = ((f_all[rows, :] - mean) * rstd * ln2w_ref[0, 0]
                    + ln2b_ref[0, 0])
            hn = jnp.maximum(fhat, 0.0)

            @pl.when(j == 0)
            def _():
                gacc[...] = jnp.zeros_like(gacc)
                nacc[...] = jnp.zeros_like(nacc)

            gacc[...] += jnp.broadcast_to(
                jnp.sum(hn, axis=0, keepdims=True), (8, C))
            nacc[...] += lax.dot_general(hn, hn, (((0,), (0,)), ((), ())),
                                         precision=_HIGHEST,
                                         preferred_element_type=jnp.float32)
            if has_next:
                hn_ref[...] = hn
                hl = jnp.dot(hn, wgn_ref[...], precision=_HIGHEST,
                             preferred_element_type=jnp.float32)
                hsn_ref[...] = hl * dinv8_ref[:, 0:1]

            @pl.when(j == NB - 1)
            def _():
                gb_ref[...] = gacc[...]
                nb_ref[...] = nacc[...]

    return body


def _tc_layer(aggp, hs, dinv8, bg_i, hprev, Wf_i, bf_i, ln1w, ln1b,
              ln2w, ln2b, Wg_next):
    has_next = Wg_next is not None
    in_specs = [
        pl.BlockSpec((2, BN, C), lambda p, j: (0, jnp.where(p == 0, j, 0), 0)),
        _row_spec_ph(0),
        _dinv8_spec_ph((0, 2) if has_next else (0,)),
        _full_spec2((1, C)),
        _row_spec_ph(1),
        _full_spec2((C, C)),
        _full_spec2((1, C)),
        _smem_spec2(), _smem_spec2(), _smem_spec2(), _smem_spec2(),
    ]
    args = [aggp, hs, dinv8, bg_i, hprev, Wf_i, bf_i, ln1w, ln1b, ln2w, ln2b]
    if has_next:
        in_specs.append(_full_spec2((C, C)))
        args.append(Wg_next)
        out_specs = [_row_spec_ph(2), _row_spec_ph(2),
                     _full_spec2((8, C)), _full_spec2((C, C))]
        out_shape = [jax.ShapeDtypeStruct((N, C), jnp.float32),
                     jax.ShapeDtypeStruct((N, C), jnp.float32),
                     jax.ShapeDtypeStruct((8, C), jnp.float32),
                     jax.ShapeDtypeStruct((C, C), jnp.float32)]
    else:
        out_specs = [_full_spec2((8, C)), _full_spec2((C, C))]
        out_shape = [jax.ShapeDtypeStruct((8, C), jnp.float32),
                     jax.ShapeDtypeStruct((C, C), jnp.float32)]
    return pl.pallas_call(
        _make_tc_layer_body(has_next),
        grid=(3, NB),
        in_specs=in_specs,
        out_specs=out_specs,
        out_shape=out_shape,
        scratch_shapes=[pltpu.VMEM((N, C), jnp.float32),
                        pltpu.VMEM((N, C), jnp.float32),
                        pltpu.SMEM((1, 1), jnp.float32),
                        pltpu.SMEM((1, 1), jnp.float32),
                        pltpu.SMEM((1, 1), jnp.float32),
                        pltpu.SMEM((1, 1), jnp.float32),
                        pltpu.VMEM((8, C), jnp.float32),
                        pltpu.VMEM((C, C), jnp.float32)],
    )(*args)


def kernel(x, edge_index, batch, W_emb, b_emb, Wg, bg, ln1w, ln1b,
           Wf, bf, ln2w, ln2b):
    src = edge_index[0]
    dst = edge_index[1]
    degp = jnp.broadcast_to(_sc_deg(dst).reshape(2, N, 1), (2, N, 8))

    h0, hs0, dinv8, gb0, nb0 = _tc_embed(x, W_emb, b_emb[None, :], Wg[0],
                                         degp)

    gbs, nbs = [gb0], [nb0]
    h, hs = h0, hs0
    for i in range(L):
        aggp = _sc_agg(src, dst, hs).reshape(2, N, C)
        outs = _tc_layer(aggp, hs, dinv8, bg[i][None, :], h,
                         Wf[i], bf[i][None, :],
                         ln1w[i].reshape(1, 1), ln1b[i].reshape(1, 1),
                         ln2w[i].reshape(1, 1), ln2b[i].reshape(1, 1),
                         Wg[i + 1] if i + 1 < L else None)
        if i + 1 < L:
            h, hs, gb_i, nb_i = outs
        else:
            gb_i, nb_i = outs
        gbs.append(gb_i)
        nbs.append(nb_i)

    gb_out = jnp.concatenate([g[0:1] for g in gbs], axis=-1)
    nb_out = jnp.stack(nbs, axis=0)[None]
    return gb_out, nb_out


# speedup vs baseline: 26.4332x; 1.0638x over previous
import functools

import jax
import jax.numpy as jnp
from jax import lax
from jax.experimental import pallas as pl
from jax.experimental.pallas import tpu as pltpu
from jax.experimental.pallas import tpu_sc as plsc

N = 10000
E = 320000
C = 128
L = 2

BN = 1000
NB = N // BN

NSC = 2
NSUB = 16
NW = NSC * NSUB
K = 128
NCH = E // K
CH_PER_W = -(-NCH // NW)
RPS = 624
RCH = 104
NRCH = RPS // RCH
TAIL0 = NSUB * RPS
TAIL = N - TAIL0

_HIGHEST = jax.lax.Precision.HIGHEST


def _sc_mesh():
    return plsc.VectorSubcoreMesh(core_axis_name="c", subcore_axis_name="s")


def _sc_deg(dst):
    @functools.partial(
        pl.kernel,
        out_type=jax.ShapeDtypeStruct((2 * N,), jnp.float32),
        mesh=_sc_mesh(),
        scratch_types=[
            pltpu.VMEM((K,), jnp.int32),
            pltpu.VMEM((K,), jnp.float32),
            pltpu.VMEM((RPS,), jnp.float32),
            pltpu.VMEM_SHARED((N,), jnp.float32),
        ],
    )
    def run(dst_hbm, out_hbm, idx_v, ones_v, zb, acc):
        c = lax.axis_index("c")
        s = lax.axis_index("s")
        wid = c * NSUB + s

        def fill_ones(i, carry):
            ones_v[pl.ds(i * 16, 16)] = jnp.full((16,), 1.0, jnp.float32)
            return carry

        lax.fori_loop(0, K // 16, fill_ones, 0)

        def fill_zero(i, carry):
            zb[pl.ds(i * 16, 16)] = jnp.zeros((16,), jnp.float32)
            return carry

        lax.fori_loop(0, RPS // 16, fill_zero, 0)
        pltpu.sync_copy(zb, acc.at[pl.ds(s * RPS, RPS)])

        @pl.when(s == 0)
        def _():
            pltpu.sync_copy(zb.at[pl.ds(0, TAIL)], acc.at[pl.ds(TAIL0, TAIL)])

        plsc.subcore_barrier()

        def chunk(i, carry):
            cid = wid + i * NW

            @pl.when(cid < NCH)
            def _():
                pltpu.sync_copy(dst_hbm.at[pl.ds(cid * K, K)], idx_v)
                pltpu.sync_copy(ones_v, acc.at[idx_v], add=True)

            return carry

        lax.fori_loop(0, CH_PER_W, chunk, 0)
        plsc.subcore_barrier()
        pltpu.sync_copy(acc.at[pl.ds(s * RPS, RPS)], zb)
        pltpu.sync_copy(zb, out_hbm.at[pl.ds(c * N + s * RPS, RPS)])

        @pl.when(s == 0)
        def _():
            pltpu.sync_copy(acc.at[pl.ds(TAIL0, TAIL)], zb.at[pl.ds(0, TAIL)])
            pltpu.sync_copy(zb.at[pl.ds(0, TAIL)],
                            out_hbm.at[pl.ds(c * N + TAIL0, TAIL)])

    return run(dst)


CHU = 80


def _sc_agg(src, dst, hs):
    @functools.partial(
        pl.kernel,
        out_type=jax.ShapeDtypeStruct((2 * N, C), jnp.float32),
        mesh=_sc_mesh(),
        scratch_types=(
            [pltpu.VMEM((K,), jnp.int32) for _ in range(8)]
            + [pltpu.VMEM((K, C), jnp.float32) for _ in range(2)]
            + [pltpu.VMEM((RCH, C), jnp.float32),
               pltpu.VMEM_SHARED((N, C), jnp.float32)]
            + [pltpu.SemaphoreType.DMA for _ in range(10)]
        ),
    )
    def run(src_hbm, dst_hbm, hs_hbm, out_hbm,
            is0, is1, is2, is3, id0, id1, id2, id3, rows0, rows1, zb, acc,
            gis0, gis1, gis2, gis3, gid0, gid1, gid2, gid3, gg0, gg1):
        isl = [is0, is1, is2, is3]
        idl = [id0, id1, id2, id3]
        rows = [rows0, rows1]
        sis = [gis0, gis1, gis2, gis3]
        sid = [gid0, gid1, gid2, gid3]
        sg = [gg0, gg1]
        c = lax.axis_index("c")
        s = lax.axis_index("s")
        wid = c * NSUB + s

        def fill_zero(i, carry):
            for j in range(C // 16):
                zb[i, pl.ds(j * 16, 16)] = jnp.zeros((16,), jnp.float32)
            return carry

        lax.fori_loop(0, RCH, fill_zero, 0)
        for k in range(NRCH):
            pltpu.sync_copy(zb, acc.at[pl.ds(s * RPS + k * RCH, RCH)])

        @pl.when(s == 0)
        def _():
            pltpu.sync_copy(zb.at[pl.ds(0, TAIL)], acc.at[pl.ds(TAIL0, TAIL)])

        plsc.subcore_barrier()

        def base_of(i):
            cid = wid + i * NW
            return jnp.where(cid < NCH, cid * K, 0), cid < NCH

        def issue_idx(i, slot):
            base, _ = base_of(i)
            pltpu.make_async_copy(src_hbm.at[pl.ds(base, K)], isl[slot],
                                  sis[slot]).start()
            pltpu.make_async_copy(dst_hbm.at[pl.ds(base, K)], idl[slot],
                                  sid[slot]).start()

        def wait_idx_s(slot):
            pltpu.make_async_copy(src_hbm.at[pl.ds(0, K)], isl[slot],
                                  sis[slot]).wait()

        def finish_chunk(i, slot, rb):
            pltpu.make_async_copy(hs_hbm.at[isl[slot]], rows[rb],
                                  sg[rb]).wait()
            pltpu.make_async_copy(dst_hbm.at[pl.ds(0, K)], idl[slot],
                                  sid[slot]).wait()
            _, valid = base_of(i)

            @pl.when(valid)
            def _():
                pltpu.sync_copy(rows[rb], acc.at[idl[slot]], add=True)

        for u in range(3):
            issue_idx(u, u)

        def step(p, carry):
            for u in range(4):
                i = 4 * p + u
                wait_idx_s(u)
                pltpu.make_async_copy(hs_hbm.at[isl[u]], rows[u % 2],
                                      sg[u % 2]).start()
                @pl.when(i >= 1)
                def _(u=u, i=i):
                    finish_chunk(i - 1, (u - 1) % 4, (u - 1) % 2)

                @pl.when(i + 3 < CHU)
                def _(u=u, i=i):
                    issue_idx(i + 3, (u + 3) % 4)

            return carry

        lax.fori_loop(0, CHU // 4, step, 0)
        finish_chunk(CHU - 1, (CHU - 1) % 4, (CHU - 1) % 2)
        plsc.subcore_barrier()
        for k in range(NRCH):
            r0 = s * RPS + k * RCH
            pltpu.sync_copy(acc.at[pl.ds(r0, RCH)],
                            out_hbm.at[pl.ds(c * N + r0, RCH)])

        @pl.when(s == 0)
        def _():
            pltpu.sync_copy(acc.at[pl.ds(TAIL0, TAIL)],
                            out_hbm.at[pl.ds(c * N + TAIL0, TAIL)])

    return run(src, dst, hs)


def _dinv_block(degp_ref):
    d = degp_ref[0, :, 0:1] + degp_ref[1, :, 0:1] + 1.0
    return lax.rsqrt(d)


def _row_spec():
    return pl.BlockSpec((BN, C), lambda j: (j, 0))


def _full_spec(shape):
    return pl.BlockSpec(shape, lambda j: tuple(0 for _ in shape))


def _degp_spec():
    return pl.BlockSpec((2, BN, 8), lambda j: (0, j, 0))


def _smem_spec():
    return pl.BlockSpec(memory_space=pltpu.SMEM)


def _tc_embed_body(x_ref, we_ref, be_ref, wg_ref,
                   h_ref, hl_ref, gb_ref, nb_ref, gacc, nacc):
    j = pl.program_id(0)
    h = jnp.dot(x_ref[...], we_ref[...], precision=_HIGHEST,
                preferred_element_type=jnp.float32) + be_ref[...]
    h_ref[...] = h
    hl_ref[...] = jnp.dot(h, wg_ref[...], precision=_HIGHEST,
                          preferred_element_type=jnp.float32)

    @pl.when(j == 0)
    def _():
        gacc[...] = jnp.zeros_like(gacc)
        nacc[...] = jnp.zeros_like(nacc)

    gacc[...] += jnp.broadcast_to(jnp.sum(h, axis=0, keepdims=True), (8, C))
    nacc[...] += lax.dot_general(h, h, (((0,), (0,)), ((), ())),
                                 precision=_HIGHEST,
                                 preferred_element_type=jnp.float32)

    @pl.when(j == NB - 1)
    def _():
        gb_ref[...] = gacc[...]
        nb_ref[...] = nacc[...]


def _tc_embed(x, W_emb, b_emb, Wg0):
    return pl.pallas_call(
        _tc_embed_body,
        grid=(NB,),
        in_specs=[_row_spec(), _full_spec((C, C)), _full_spec((1, C)),
                  _full_spec((C, C))],
        out_specs=[_row_spec(), _row_spec(),
                   _full_spec((8, C)), _full_spec((C, C))],
        out_shape=[jax.ShapeDtypeStruct((N, C), jnp.float32),
                   jax.ShapeDtypeStruct((N, C), jnp.float32),
                   jax.ShapeDtypeStruct((8, C), jnp.float32),
                   jax.ShapeDtypeStruct((C, C), jnp.float32)],
        scratch_shapes=[pltpu.VMEM((8, C), jnp.float32),
                        pltpu.VMEM((C, C), jnp.float32)],
    )(x, W_emb, b_emb, Wg0)


def _tc_scale_body(hl_ref, degp_ref, hs_ref, dinv8_ref):
    dinv = _dinv_block(degp_ref)
    dinv8_ref[...] = jnp.broadcast_to(dinv, (BN, 8))
    hs_ref[...] = hl_ref[...] * dinv


def _tc_scale(hl, degp):
    return pl.pallas_call(
        _tc_scale_body,
        grid=(NB,),
        in_specs=[_row_spec(), _degp_spec()],
        out_specs=[_row_spec(), pl.BlockSpec((BN, 8), lambda j: (j, 0))],
        out_shape=[jax.ShapeDtypeStruct((N, C), jnp.float32),
                   jax.ShapeDtypeStruct((N, 8), jnp.float32)],
    )(hl, degp)


_INV_NC = 1.0 / (N * C)
_EPS = 1e-5


def _dinv8_spec_ph(phases):
    def imap(p, j):
        use = (p == phases[0])
        for q in phases[1:]:
            use = use | (p == q)
        return (jnp.where(use, j, 0), 0)
    return pl.BlockSpec((BN, 8), imap)


def _row_spec_ph(phase):
    return pl.BlockSpec((BN, C), lambda p, j: (jnp.where(p == phase, j, 0), 0))


def _full_spec2(shape):
    return pl.BlockSpec(shape, lambda p, j: tuple(0 for _ in shape))


def _smem_spec2():
    return pl.BlockSpec(memory_space=pltpu.SMEM)


def _make_tc_layer_body(has_next):
    def body(*refs):
        if has_next:
            (aggp_ref, hs_ref, dinv8_ref, bg_ref, hp_ref, wf_ref, bf_ref,
             ln1w_ref, ln1b_ref, ln2w_ref, ln2b_ref, wgn_ref,
             hn_ref, hsn_ref, gb_ref, nb_ref,
             m_all, f_all, sm1, sm2, sf1, sf2, gacc, nacc) = refs
        else:
            (aggp_ref, hs_ref, dinv8_ref, bg_ref, hp_ref, wf_ref, bf_ref,
             ln1w_ref, ln1b_ref, ln2w_ref, ln2b_ref,
             gb_ref, nb_ref,
             m_all, f_all, sm1, sm2, sf1, sf2, gacc, nacc) = refs
        p = pl.program_id(0)
        j = pl.program_id(1)
        rows = pl.ds(j * BN, BN)

        @pl.when(p == 0)
        def _():
            dinv = dinv8_ref[:, 0:1]
            m = (aggp_ref[0] + aggp_ref[1] + hs_ref[...]) * dinv + bg_ref[...]
            m_all[rows, :] = m

            @pl.when(j == 0)
            def _():
                sm1[0, 0] = 0.0
                sm2[0, 0] = 0.0

            sm1[0, 0] += jnp.sum(m)
            sm2[0, 0] += jnp.sum(m * m)

        @pl.when(p == 1)
        def _():
            mean = sm1[0, 0] * _INV_NC
            var = sm2[0, 0] * _INV_NC - mean * mean
            rstd = lax.rsqrt(var + _EPS)
            mhat = ((m_all[rows, :] - mean) * rstd * ln1w_ref[0, 0]
                    + ln1b_ref[0, 0])
            hmid = hp_ref[...] + jnp.maximum(mhat, 0.0)
            f = jnp.dot(hmid, wf_ref[...], precision=_HIGHEST,
                        preferred_element_type=jnp.float32) + bf_ref[...]
            f_all[rows, :] = f

            @pl.when(j == 0)
            def _():
                sf1[0, 0] = 0.0
                sf2[0, 0] = 0.0

            sf1[0, 0] += jnp.sum(f)
            sf2[0, 0] += jnp.sum(f * f)

        @pl.when(p == 2)
        def _():
            mean = sf1[0, 0] * _INV_NC
            var = sf2[0, 0] * _INV_NC - mean * mean
            rstd = lax.rsqrt(var + _EPS)
            fhat = ((f_all[rows, :] - mean) * rstd * ln2w_ref[0, 0]
                    + ln2b_ref[0, 0])
            hn = jnp.maximum(fhat, 0.0)

            @pl.when(j == 0)
            def _():
                gacc[...] = jnp.zeros_like(gacc)
                nacc[...] = jnp.zeros_like(nacc)

            gacc[...] += jnp.broadcast_to(
                jnp.sum(hn, axis=0, keepdims=True), (8, C))
            nacc[...] += lax.dot_general(hn, hn, (((0,), (0,)), ((), ())),
                                         precision=_HIGHEST,
                                         preferred_element_type=jnp.float32)
            if has_next:
                hn_ref[...] = hn
                hl = jnp.dot(hn, wgn_ref[...], precision=_HIGHEST,
                             preferred_element_type=jnp.float32)
                hsn_ref[...] = hl * dinv8_ref[:, 0:1]

            @pl.when(j == NB - 1)
            def _():
                gb_ref[...] = gacc[...]
                nb_ref[...] = nacc[...]

    return body


def _tc_layer(aggp, hs, dinv8, bg_i, hprev, Wf_i, bf_i, ln1w, ln1b,
              ln2w, ln2b, Wg_next):
    has_next = Wg_next is not None
    in_specs = [
        pl.BlockSpec((2, BN, C), lambda p, j: (0, jnp.where(p == 0, j, 0), 0)),
        _row_spec_ph(0),
        _dinv8_spec_ph((0, 2) if has_next else (0,)),
        _full_spec2((1, C)),
        _row_spec_ph(1),
        _full_spec2((C, C)),
        _full_spec2((1, C)),
        _smem_spec2(), _smem_spec2(), _smem_spec2(), _smem_spec2(),
    ]
    args = [aggp, hs, dinv8, bg_i, hprev, Wf_i, bf_i, ln1w, ln1b, ln2w, ln2b]
    if has_next:
        in_specs.append(_full_spec2((C, C)))
        args.append(Wg_next)
        out_specs = [_row_spec_ph(2), _row_spec_ph(2),
                     _full_spec2((8, C)), _full_spec2((C, C))]
        out_shape = [jax.ShapeDtypeStruct((N, C), jnp.float32),
                     jax.ShapeDtypeStruct((N, C), jnp.float32),
                     jax.ShapeDtypeStruct((8, C), jnp.float32),
                     jax.ShapeDtypeStruct((C, C), jnp.float32)]
    else:
        out_specs = [_full_spec2((8, C)), _full_spec2((C, C))]
        out_shape = [jax.ShapeDtypeStruct((8, C), jnp.float32),
                     jax.ShapeDtypeStruct((C, C), jnp.float32)]
    return pl.pallas_call(
        _make_tc_layer_body(has_next),
        grid=(3, NB),
        in_specs=in_specs,
        out_specs=out_specs,
        out_shape=out_shape,
        scratch_shapes=[pltpu.VMEM((N, C), jnp.float32),
                        pltpu.VMEM((N, C), jnp.float32),
                        pltpu.SMEM((1, 1), jnp.float32),
                        pltpu.SMEM((1, 1), jnp.float32),
                        pltpu.SMEM((1, 1), jnp.float32),
                        pltpu.SMEM((1, 1), jnp.float32),
                        pltpu.VMEM((8, C), jnp.float32),
                        pltpu.VMEM((C, C), jnp.float32)],
    )(*args)


def kernel(x, edge_index, batch, W_emb, b_emb, Wg, bg, ln1w, ln1b,
           Wf, bf, ln2w, ln2b):
    src = edge_index[0]
    dst = edge_index[1]
    degp = jnp.broadcast_to(_sc_deg(dst).reshape(2, N, 1), (2, N, 8))

    h0, hl0, gb0, nb0 = _tc_embed(x, W_emb, b_emb[None, :], Wg[0])
    hs0, dinv8 = _tc_scale(hl0, degp)

    gbs, nbs = [gb0], [nb0]
    h, hs = h0, hs0
    for i in range(L):
        aggp = _sc_agg(src, dst, hs).reshape(2, N, C)
        outs = _tc_layer(aggp, hs, dinv8, bg[i][None, :], h,
                         Wf[i], bf[i][None, :],
                         ln1w[i].reshape(1, 1), ln1b[i].reshape(1, 1),
                         ln2w[i].reshape(1, 1), ln2b[i].reshape(1, 1),
                         Wg[i + 1] if i + 1 < L else None)
        if i + 1 < L:
            h, hs, gb_i, nb_i = outs
        else:
            gb_i, nb_i = outs
        gbs.append(gb_i)
        nbs.append(nb_i)

    gb_out = jnp.concatenate([g[0:1] for g in gbs], axis=-1)
    nb_out = jnp.stack(nbs, axis=0)[None]
    return gb_out, nb_out


# speedup vs baseline: 26.8785x; 1.0168x over previous
import functools

import jax
import jax.numpy as jnp
from jax import lax
from jax.experimental import pallas as pl
from jax.experimental.pallas import tpu as pltpu
from jax.experimental.pallas import tpu_sc as plsc

N = 10000
E = 320000
C = 128
L = 2

BN = 1000
NB = N // BN

NSC = 2
NSUB = 16
NW = NSC * NSUB
K = 128
NCH = E // K
CH_PER_W = -(-NCH // NW)
RPS = 624
RCH = 104
NRCH = RPS // RCH
TAIL0 = NSUB * RPS
TAIL = N - TAIL0

_HIGHEST = jax.lax.Precision.HIGHEST


def _sc_mesh():
    return plsc.VectorSubcoreMesh(core_axis_name="c", subcore_axis_name="s")


def _sc_deg(dst):
    @functools.partial(
        pl.kernel,
        out_type=jax.ShapeDtypeStruct((2 * N,), jnp.float32),
        mesh=_sc_mesh(),
        scratch_types=[
            pltpu.VMEM((K,), jnp.int32),
            pltpu.VMEM((K,), jnp.int32),
            pltpu.VMEM((K,), jnp.int32),
            pltpu.VMEM((K,), jnp.int32),
            pltpu.VMEM((K,), jnp.float32),
            pltpu.VMEM((RPS,), jnp.float32),
            pltpu.VMEM_SHARED((N,), jnp.float32),
            pltpu.SemaphoreType.DMA,
            pltpu.SemaphoreType.DMA,
            pltpu.SemaphoreType.DMA,
            pltpu.SemaphoreType.DMA,
        ],
    )
    def run(dst_hbm, out_hbm, ix0, ix1, ix2, ix3, ones_v, zb, acc,
            sx0, sx1, sx2, sx3):
        ixl = [ix0, ix1, ix2, ix3]
        sxl = [sx0, sx1, sx2, sx3]
        c = lax.axis_index("c")
        s = lax.axis_index("s")
        wid = c * NSUB + s

        def fill_ones(i, carry):
            ones_v[pl.ds(i * 16, 16)] = jnp.full((16,), 1.0, jnp.float32)
            return carry

        lax.fori_loop(0, K // 16, fill_ones, 0)

        def fill_zero(i, carry):
            zb[pl.ds(i * 16, 16)] = jnp.zeros((16,), jnp.float32)
            return carry

        lax.fori_loop(0, RPS // 16, fill_zero, 0)
        pltpu.sync_copy(zb, acc.at[pl.ds(s * RPS, RPS)])

        @pl.when(s == 0)
        def _():
            pltpu.sync_copy(zb.at[pl.ds(0, TAIL)], acc.at[pl.ds(TAIL0, TAIL)])

        plsc.subcore_barrier()

        def base_of(i):
            cid = wid + i * NW
            return jnp.where(cid < NCH, cid * K, 0), cid < NCH

        def issue_idx(i, slot):
            base, _ = base_of(i)
            pltpu.make_async_copy(dst_hbm.at[pl.ds(base, K)], ixl[slot],
                                  sxl[slot]).start()

        for u in range(3):
            issue_idx(u, u)

        def step(p, carry):
            for u in range(4):
                i = 4 * p + u
                pltpu.make_async_copy(dst_hbm.at[pl.ds(0, K)], ixl[u],
                                      sxl[u]).wait()
                _, valid = base_of(i)

                @pl.when(valid)
                def _(u=u):
                    pltpu.sync_copy(ones_v, acc.at[ixl[u]], add=True)

                @pl.when(i + 3 < CHU)
                def _(u=u, i=i):
                    issue_idx(i + 3, (u + 3) % 4)

            return carry

        lax.fori_loop(0, CHU // 4, step, 0)
        plsc.subcore_barrier()
        pltpu.sync_copy(acc.at[pl.ds(s * RPS, RPS)], zb)
        pltpu.sync_copy(zb, out_hbm.at[pl.ds(c * N + s * RPS, RPS)])

        @pl.when(s == 0)
        def _():
            pltpu.sync_copy(acc.at[pl.ds(TAIL0, TAIL)], zb.at[pl.ds(0, TAIL)])
            pltpu.sync_copy(zb.at[pl.ds(0, TAIL)],
                            out_hbm.at[pl.ds(c * N + TAIL0, TAIL)])

    return run(dst)


CHU = 80


def _sc_agg(src, dst, hs):
    @functools.partial(
        pl.kernel,
        out_type=jax.ShapeDtypeStruct((2 * N, C), jnp.float32),
        mesh=_sc_mesh(),
        scratch_types=(
            [pltpu.VMEM((K,), jnp.int32) for _ in range(8)]
            + [pltpu.VMEM((K, C), jnp.float32) for _ in range(2)]
            + [pltpu.VMEM((RCH, C), jnp.float32),
               pltpu.VMEM_SHARED((N, C), jnp.float32)]
            + [pltpu.SemaphoreType.DMA for _ in range(10)]
        ),
    )
    def run(src_hbm, dst_hbm, hs_hbm, out_hbm,
            is0, is1, is2, is3, id0, id1, id2, id3, rows0, rows1, zb, acc,
            gis0, gis1, gis2, gis3, gid0, gid1, gid2, gid3, gg0, gg1):
        isl = [is0, is1, is2, is3]
        idl = [id0, id1, id2, id3]
        rows = [rows0, rows1]
        sis = [gis0, gis1, gis2, gis3]
        sid = [gid0, gid1, gid2, gid3]
        sg = [gg0, gg1]
        c = lax.axis_index("c")
        s = lax.axis_index("s")
        wid = c * NSUB + s

        def fill_zero(i, carry):
            for j in range(C // 16):
                zb[i, pl.ds(j * 16, 16)] = jnp.zeros((16,), jnp.float32)
            return carry

        lax.fori_loop(0, RCH, fill_zero, 0)
        for k in range(NRCH):
            pltpu.sync_copy(zb, acc.at[pl.ds(s * RPS + k * RCH, RCH)])

        @pl.when(s == 0)
        def _():
            pltpu.sync_copy(zb.at[pl.ds(0, TAIL)], acc.at[pl.ds(TAIL0, TAIL)])

        plsc.subcore_barrier()

        def base_of(i):
            cid = wid + i * NW
            return jnp.where(cid < NCH, cid * K, 0), cid < NCH

        def issue_idx(i, slot):
            base, _ = base_of(i)
            pltpu.make_async_copy(src_hbm.at[pl.ds(base, K)], isl[slot],
                                  sis[slot]).start()
            pltpu.make_async_copy(dst_hbm.at[pl.ds(base, K)], idl[slot],
                                  sid[slot]).start()

        def wait_idx_s(slot):
            pltpu.make_async_copy(src_hbm.at[pl.ds(0, K)], isl[slot],
                                  sis[slot]).wait()

        def finish_chunk(i, slot, rb):
            pltpu.make_async_copy(hs_hbm.at[isl[slot]], rows[rb],
                                  sg[rb]).wait()
            pltpu.make_async_copy(dst_hbm.at[pl.ds(0, K)], idl[slot],
                                  sid[slot]).wait()
            _, valid = base_of(i)

            @pl.when(valid)
            def _():
                pltpu.sync_copy(rows[rb], acc.at[idl[slot]], add=True)

        for u in range(3):
            issue_idx(u, u)

        def step(p, carry):
            for u in range(4):
                i = 4 * p + u
                wait_idx_s(u)
                pltpu.make_async_copy(hs_hbm.at[isl[u]], rows[u % 2],
                                      sg[u % 2]).start()
                @pl.when(i >= 1)
                def _(u=u, i=i):
                    finish_chunk(i - 1, (u - 1) % 4, (u - 1) % 2)

                @pl.when(i + 3 < CHU)
                def _(u=u, i=i):
                    issue_idx(i + 3, (u + 3) % 4)

            return carry

        lax.fori_loop(0, CHU // 4, step, 0)
        finish_chunk(CHU - 1, (CHU - 1) % 4, (CHU - 1) % 2)
        plsc.subcore_barrier()
        for k in range(NRCH):
            r0 = s * RPS + k * RCH
            pltpu.sync_copy(acc.at[pl.ds(r0, RCH)],
                            out_hbm.at[pl.ds(c * N + r0, RCH)])

        @pl.when(s == 0)
        def _():
            pltpu.sync_copy(acc.at[pl.ds(TAIL0, TAIL)],
                            out_hbm.at[pl.ds(c * N + TAIL0, TAIL)])

    return run(src, dst, hs)


def _dinv_block(degp_ref):
    d = degp_ref[0, :, 0:1] + degp_ref[1, :, 0:1] + 1.0
    return lax.rsqrt(d)


def _row_spec():
    return pl.BlockSpec((BN, C), lambda j: (j, 0))


def _full_spec(shape):
    return pl.BlockSpec(shape, lambda j: tuple(0 for _ in shape))


def _degp_spec():
    return pl.BlockSpec((2, BN, 8), lambda j: (0, j, 0))


def _smem_spec():
    return pl.BlockSpec(memory_space=pltpu.SMEM)


def _tc_embed_body(x_ref, we_ref, be_ref, wg_ref,
                   h_ref, hl_ref, gb_ref, nb_ref, gacc, nacc):
    j = pl.program_id(0)
    h = jnp.dot(x_ref[...], we_ref[...], precision=_HIGHEST,
                preferred_element_type=jnp.float32) + be_ref[...]
    h_ref[...] = h
    hl_ref[...] = jnp.dot(h, wg_ref[...], precision=_HIGHEST,
                          preferred_element_type=jnp.float32)

    @pl.when(j == 0)
    def _():
        gacc[...] = jnp.zeros_like(gacc)
        nacc[...] = jnp.zeros_like(nacc)

    gacc[...] += jnp.broadcast_to(jnp.sum(h, axis=0, keepdims=True), (8, C))
    nacc[...] += lax.dot_general(h, h, (((0,), (0,)), ((), ())),
                                 precision=_HIGHEST,
                                 preferred_element_type=jnp.float32)

    @pl.when(j == NB - 1)
    def _():
        gb_ref[...] = gacc[...]
        nb_ref[...] = nacc[...]


def _tc_embed(x, W_emb, b_emb, Wg0):
    return pl.pallas_call(
        _tc_embed_body,
        grid=(NB,),
        in_specs=[_row_spec(), _full_spec((C, C)), _full_spec((1, C)),
                  _full_spec((C, C))],
        out_specs=[_row_spec(), _row_spec(),
                   _full_spec((8, C)), _full_spec((C, C))],
        out_shape=[jax.ShapeDtypeStruct((N, C), jnp.float32),
                   jax.ShapeDtypeStruct((N, C), jnp.float32),
                   jax.ShapeDtypeStruct((8, C), jnp.float32),
                   jax.ShapeDtypeStruct((C, C), jnp.float32)],
        scratch_shapes=[pltpu.VMEM((8, C), jnp.float32),
                        pltpu.VMEM((C, C), jnp.float32)],
    )(x, W_emb, b_emb, Wg0)


def _tc_scale_body(hl_ref, degp_ref, hs_ref, dinv8_ref):
    dinv = _dinv_block(degp_ref)
    dinv8_ref[...] = jnp.broadcast_to(dinv, (BN, 8))
    hs_ref[...] = hl_ref[...] * dinv


def _tc_scale(hl, degp):
    return pl.pallas_call(
        _tc_scale_body,
        grid=(NB,),
        in_specs=[_row_spec(), _degp_spec()],
        out_specs=[_row_spec(), pl.BlockSpec((BN, 8), lambda j: (j, 0))],
        out_shape=[jax.ShapeDtypeStruct((N, C), jnp.float32),
                   jax.ShapeDtypeStruct((N, 8), jnp.float32)],
    )(hl, degp)


_INV_NC = 1.0 / (N * C)
_EPS = 1e-5


def _dinv8_spec_ph(phases):
    def imap(p, j):
        use = (p == phases[0])
        for q in phases[1:]:
            use = use | (p == q)
        return (jnp.where(use, j, 0), 0)
    return pl.BlockSpec((BN, 8), imap)


def _row_spec_ph(phase):
    return pl.BlockSpec((BN, C), lambda p, j: (jnp.where(p == phase, j, 0), 0))


def _full_spec2(shape):
    return pl.BlockSpec(shape, lambda p, j: tuple(0 for _ in shape))


def _smem_spec2():
    return pl.BlockSpec(memory_space=pltpu.SMEM)


def _make_tc_layer_body(has_next):
    def body(*refs):
        if has_next:
            (aggp_ref, hs_ref, dinv8_ref, bg_ref, hp_ref, wf_ref, bf_ref,
             ln1w_ref, ln1b_ref, ln2w_ref, ln2b_ref, wgn_ref,
             hn_ref, hsn_ref, gb_ref, nb_ref,
             m_all, f_all, sm1, sm2, sf1, sf2, gacc, nacc) = refs
        else:
            (aggp_ref, hs_ref, dinv8_ref, bg_ref, hp_ref, wf_ref, bf_ref,
             ln1w_ref, ln1b_ref, ln2w_ref, ln2b_ref,
             gb_ref, nb_ref,
             m_all, f_all, sm1, sm2, sf1, sf2, gacc, nacc) = refs
        p = pl.program_id(0)
        j = pl.program_id(1)
        rows = pl.ds(j * BN, BN)

        @pl.when(p == 0)
        def _():
            dinv = dinv8_ref[:, 0:1]
            m = (aggp_ref[0] + aggp_ref[1] + hs_ref[...]) * dinv + bg_ref[...]
            m_all[rows, :] = m

            @pl.when(j == 0)
            def _():
                sm1[0, 0] = 0.0
                sm2[0, 0] = 0.0

            sm1[0, 0] += jnp.sum(m)
            sm2[0, 0] += jnp.sum(m * m)

        @pl.when(p == 1)
        def _():
            mean = sm1[0, 0] * _INV_NC
            var = sm2[0, 0] * _INV_NC - mean * mean
            rstd = lax.rsqrt(var + _EPS)
            mhat = ((m_all[rows, :] - mean) * rstd * ln1w_ref[0, 0]
                    + ln1b_ref[0, 0])
            hmid = hp_ref[...] + jnp.maximum(mhat, 0.0)
            f = jnp.dot(hmid, wf_ref[...], precision=_HIGHEST,
                        preferred_element_type=jnp.float32) + bf_ref[...]
            f_all[rows, :] = f

            @pl.when(j == 0)
            def _():
                sf1[0, 0] = 0.0
                sf2[0, 0] = 0.0

            sf1[0, 0] += jnp.sum(f)
            sf2[0, 0] += jnp.sum(f * f)

        @pl.when(p == 2)
        def _():
            mean = sf1[0, 0] * _INV_NC
            var = sf2[0, 0] * _INV_NC - mean * mean
            rstd = lax.rsqrt(var + _EPS)
            fhat = ((f_all[rows, :] - mean) * rstd * ln2w_ref[0, 0]
                    + ln2b_ref[0, 0])
            hn = jnp.maximum(fhat, 0.0)

            @pl.when(j == 0)
            def _():
                gacc[...] = jnp.zeros_like(gacc)
                nacc[...] = jnp.zeros_like(nacc)

            gacc[...] += jnp.broadcast_to(
                jnp.sum(hn, axis=0, keepdims=True), (8, C))
            nacc[...] += lax.dot_general(hn, hn, (((0,), (0,)), ((), ())),
                                         precision=_HIGHEST,
                                         preferred_element_type=jnp.float32)
            if has_next:
                hn_ref[...] = hn
                hl = jnp.dot(hn, wgn_ref[...], precision=_HIGHEST,
                             preferred_element_type=jnp.float32)
                hsn_ref[...] = hl * dinv8_ref[:, 0:1]

            @pl.when(j == NB - 1)
            def _():
                gb_ref[...] = gacc[...]
                nb_ref[...] = nacc[...]

    return body


def _tc_layer(aggp, hs, dinv8, bg_i, hprev, Wf_i, bf_i, ln1w, ln1b,
              ln2w, ln2b, Wg_next):
    has_next = Wg_next is not None
    in_specs = [
        pl.BlockSpec((2, BN, C), lambda p, j: (0, jnp.where(p == 0, j, 0), 0)),
        _row_spec_ph(0),
        _dinv8_spec_ph((0, 2) if has_next else (0,)),
        _full_spec2((1, C)),
        _row_spec_ph(1),
        _full_spec2((C, C)),
        _full_spec2((1, C)),
        _smem_spec2(), _smem_spec2(), _smem_spec2(), _smem_spec2(),
    ]
    args = [aggp, hs, dinv8, bg_i, hprev, Wf_i, bf_i, ln1w, ln1b, ln2w, ln2b]
    if has_next:
        in_specs.append(_full_spec2((C, C)))
        args.append(Wg_next)
        out_specs = [_row_spec_ph(2), _row_spec_ph(2),
                     _full_spec2((8, C)), _full_spec2((C, C))]
        out_shape = [jax.ShapeDtypeStruct((N, C), jnp.float32),
                     jax.ShapeDtypeStruct((N, C), jnp.float32),
                     jax.ShapeDtypeStruct((8, C), jnp.float32),
                     jax.ShapeDtypeStruct((C, C), jnp.float32)]
    else:
        out_specs = [_full_spec2((8, C)), _full_spec2((C, C))]
        out_shape = [jax.ShapeDtypeStruct((8, C), jnp.float32),
                     jax.ShapeDtypeStruct((C, C), jnp.float32)]
    return pl.pallas_call(
        _make_tc_layer_body(has_next),
        grid=(3, NB),
        in_specs=in_specs,
        out_specs=out_specs,
        out_shape=out_shape,
        scratch_shapes=[pltpu.VMEM((N, C), jnp.float32),
                        pltpu.VMEM((N, C), jnp.float32),
                        pltpu.SMEM((1, 1), jnp.float32),
                        pltpu.SMEM((1, 1), jnp.float32),
                        pltpu.SMEM((1, 1), jnp.float32),
                        pltpu.SMEM((1, 1), jnp.float32),
                        pltpu.VMEM((8, C), jnp.float32),
                        pltpu.VMEM((C, C), jnp.float32)],
    )(*args)


def kernel(x, edge_index, batch, W_emb, b_emb, Wg, bg, ln1w, ln1b,
           Wf, bf, ln2w, ln2b):
    src = edge_index[0]
    dst = edge_index[1]
    degp = jnp.broadcast_to(_sc_deg(dst).reshape(2, N, 1), (2, N, 8))

    h0, hl0, gb0, nb0 = _tc_embed(x, W_emb, b_emb[None, :], Wg[0])
    hs0, dinv8 = _tc_scale(hl0, degp)

    gbs, nbs = [gb0], [nb0]
    h, hs = h0, hs0
    for i in range(L):
        aggp = _sc_agg(src, dst, hs).reshape(2, N, C)
        outs = _tc_layer(aggp, hs, dinv8, bg[i][None, :], h,
                         Wf[i], bf[i][None, :],
                         ln1w[i].reshape(1, 1), ln1b[i].reshape(1, 1),
                         ln2w[i].reshape(1, 1), ln2b[i].reshape(1, 1),
                         Wg[i + 1] if i + 1 < L else None)
        if i + 1 < L:
            h, hs, gb_i, nb_i = outs
        else:
            gb_i, nb_i = outs
        gbs.append(gb_i)
        nbs.append(nb_i)

    gb_out = jnp.concatenate([g[0:1] for g in gbs], axis=-1)
    nb_out = jnp.stack(nbs, axis=0)[None]
    return gb_out, nb_out


# speedup vs baseline: 29.9217x; 1.1132x over previous
import functools

import jax
import jax.numpy as jnp
from jax import lax
from jax.experimental import pallas as pl
from jax.experimental.pallas import tpu as pltpu
from jax.experimental.pallas import tpu_sc as plsc

N = 10000
E = 320000
C = 128
L = 2

BN = 2000
NB = N // BN

NSC = 2
NSUB = 16
NW = NSC * NSUB
K = 128
NCH = E // K
CH_PER_W = -(-NCH // NW)
RPS = 624
RCH = 104
NRCH = RPS // RCH
TAIL0 = NSUB * RPS
TAIL = N - TAIL0

_HIGHEST = jax.lax.Precision.HIGHEST


def _sc_mesh():
    return plsc.VectorSubcoreMesh(core_axis_name="c", subcore_axis_name="s")


def _sc_deg(dst):
    @functools.partial(
        pl.kernel,
        out_type=jax.ShapeDtypeStruct((2 * N,), jnp.float32),
        mesh=_sc_mesh(),
        scratch_types=[
            pltpu.VMEM((K,), jnp.int32),
            pltpu.VMEM((K,), jnp.int32),
            pltpu.VMEM((K,), jnp.int32),
            pltpu.VMEM((K,), jnp.int32),
            pltpu.VMEM((K,), jnp.float32),
            pltpu.VMEM((RPS,), jnp.float32),
            pltpu.VMEM_SHARED((N,), jnp.float32),
            pltpu.SemaphoreType.DMA,
            pltpu.SemaphoreType.DMA,
            pltpu.SemaphoreType.DMA,
            pltpu.SemaphoreType.DMA,
        ],
    )
    def run(dst_hbm, out_hbm, ix0, ix1, ix2, ix3, ones_v, zb, acc,
            sx0, sx1, sx2, sx3):
        ixl = [ix0, ix1, ix2, ix3]
        sxl = [sx0, sx1, sx2, sx3]
        c = lax.axis_index("c")
        s = lax.axis_index("s")
        wid = c * NSUB + s

        def fill_ones(i, carry):
            ones_v[pl.ds(i * 16, 16)] = jnp.full((16,), 1.0, jnp.float32)
            return carry

        lax.fori_loop(0, K // 16, fill_ones, 0)

        def fill_zero(i, carry):
            zb[pl.ds(i * 16, 16)] = jnp.zeros((16,), jnp.float32)
            return carry

        lax.fori_loop(0, RPS // 16, fill_zero, 0)
        pltpu.sync_copy(zb, acc.at[pl.ds(s * RPS, RPS)])

        @pl.when(s == 0)
        def _():
            pltpu.sync_copy(zb.at[pl.ds(0, TAIL)], acc.at[pl.ds(TAIL0, TAIL)])

        plsc.subcore_barrier()

        def base_of(i):
            cid = wid + i * NW
            return jnp.where(cid < NCH, cid * K, 0), cid < NCH

        def issue_idx(i, slot):
            base, _ = base_of(i)
            pltpu.make_async_copy(dst_hbm.at[pl.ds(base, K)], ixl[slot],
                                  sxl[slot]).start()

        for u in range(3):
            issue_idx(u, u)

        def step(p, carry):
            for u in range(4):
                i = 4 * p + u
                pltpu.make_async_copy(dst_hbm.at[pl.ds(0, K)], ixl[u],
                                      sxl[u]).wait()
                _, valid = base_of(i)

                @pl.when(valid)
                def _(u=u):
                    pltpu.sync_copy(ones_v, acc.at[ixl[u]], add=True)

                @pl.when(i + 3 < CHU)
                def _(u=u, i=i):
                    issue_idx(i + 3, (u + 3) % 4)

            return carry

        lax.fori_loop(0, CHU // 4, step, 0)
        plsc.subcore_barrier()
        pltpu.sync_copy(acc.at[pl.ds(s * RPS, RPS)], zb)
        pltpu.sync_copy(zb, out_hbm.at[pl.ds(c * N + s * RPS, RPS)])

        @pl.when(s == 0)
        def _():
            pltpu.sync_copy(acc.at[pl.ds(TAIL0, TAIL)], zb.at[pl.ds(0, TAIL)])
            pltpu.sync_copy(zb.at[pl.ds(0, TAIL)],
                            out_hbm.at[pl.ds(c * N + TAIL0, TAIL)])

    return run(dst)


CHU = 80


def _sc_agg(src, dst, hs):
    @functools.partial(
        pl.kernel,
        out_type=jax.ShapeDtypeStruct((2 * N, C), jnp.float32),
        mesh=_sc_mesh(),
        scratch_types=(
            [pltpu.VMEM((K,), jnp.int32) for _ in range(8)]
            + [pltpu.VMEM((K, C), jnp.float32) for _ in range(2)]
            + [pltpu.VMEM((RCH, C), jnp.float32),
               pltpu.VMEM_SHARED((N, C), jnp.float32)]
            + [pltpu.SemaphoreType.DMA for _ in range(10)]
        ),
    )
    def run(src_hbm, dst_hbm, hs_hbm, out_hbm,
            is0, is1, is2, is3, id0, id1, id2, id3, rows0, rows1, zb, acc,
            gis0, gis1, gis2, gis3, gid0, gid1, gid2, gid3, gg0, gg1):
        isl = [is0, is1, is2, is3]
        idl = [id0, id1, id2, id3]
        rows = [rows0, rows1]
        sis = [gis0, gis1, gis2, gis3]
        sid = [gid0, gid1, gid2, gid3]
        sg = [gg0, gg1]
        c = lax.axis_index("c")
        s = lax.axis_index("s")
        wid = c * NSUB + s

        def fill_zero(i, carry):
            for j in range(C // 16):
                zb[i, pl.ds(j * 16, 16)] = jnp.zeros((16,), jnp.float32)
            return carry

        lax.fori_loop(0, RCH, fill_zero, 0)
        for k in range(NRCH):
            pltpu.sync_copy(zb, acc.at[pl.ds(s * RPS + k * RCH, RCH)])

        @pl.when(s == 0)
        def _():
            pltpu.sync_copy(zb.at[pl.ds(0, TAIL)], acc.at[pl.ds(TAIL0, TAIL)])

        plsc.subcore_barrier()

        def base_of(i):
            cid = wid + i * NW
            return jnp.where(cid < NCH, cid * K, 0), cid < NCH

        def issue_idx(i, slot):
            base, _ = base_of(i)
            pltpu.make_async_copy(src_hbm.at[pl.ds(base, K)], isl[slot],
                                  sis[slot]).start()
            pltpu.make_async_copy(dst_hbm.at[pl.ds(base, K)], idl[slot],
                                  sid[slot]).start()

        def wait_idx_s(slot):
            pltpu.make_async_copy(src_hbm.at[pl.ds(0, K)], isl[slot],
                                  sis[slot]).wait()

        def finish_chunk(i, slot, rb):
            pltpu.make_async_copy(hs_hbm.at[isl[slot]], rows[rb],
                                  sg[rb]).wait()
            pltpu.make_async_copy(dst_hbm.at[pl.ds(0, K)], idl[slot],
                                  sid[slot]).wait()
            _, valid = base_of(i)

            @pl.when(valid)
            def _():
                pltpu.sync_copy(rows[rb], acc.at[idl[slot]], add=True)

        for u in range(3):
            issue_idx(u, u)

        def step(p, carry):
            for u in range(4):
                i = 4 * p + u
                wait_idx_s(u)
                pltpu.make_async_copy(hs_hbm.at[isl[u]], rows[u % 2],
                                      sg[u % 2]).start()
                @pl.when(i >= 1)
                def _(u=u, i=i):
                    finish_chunk(i - 1, (u - 1) % 4, (u - 1) % 2)

                @pl.when(i + 3 < CHU)
                def _(u=u, i=i):
                    issue_idx(i + 3, (u + 3) % 4)

            return carry

        lax.fori_loop(0, CHU // 4, step, 0)
        finish_chunk(CHU - 1, (CHU - 1) % 4, (CHU - 1) % 2)
        plsc.subcore_barrier()
        pltpu.sync_copy(acc.at[pl.ds(s * RPS, RPS)],
                        out_hbm.at[pl.ds(c * N + s * RPS, RPS)])

        @pl.when(s == 0)
        def _():
            pltpu.sync_copy(acc.at[pl.ds(TAIL0, TAIL)],
                            out_hbm.at[pl.ds(c * N + TAIL0, TAIL)])

    return run(src, dst, hs)


def _dinv_block(degp_ref):
    d = degp_ref[0, :, 0:1] + degp_ref[1, :, 0:1] + 1.0
    return lax.rsqrt(d)


def _row_spec():
    return pl.BlockSpec((BN, C), lambda j: (j, 0))


def _full_spec(shape):
    return pl.BlockSpec(shape, lambda j: tuple(0 for _ in shape))


def _degp_spec():
    return pl.BlockSpec((2, BN, 8), lambda j: (0, j, 0))


def _smem_spec():
    return pl.BlockSpec(memory_space=pltpu.SMEM)


def _tc_embed_body(x_ref, we_ref, be_ref, wg_ref,
                   h_ref, hl_ref, gb_ref, nb_ref, gacc, nacc):
    j = pl.program_id(0)
    h = jnp.dot(x_ref[...], we_ref[...], precision=_HIGHEST,
                preferred_element_type=jnp.float32) + be_ref[...]
    h_ref[...] = h
    hl_ref[...] = jnp.dot(h, wg_ref[...], precision=_HIGHEST,
                          preferred_element_type=jnp.float32)

    @pl.when(j == 0)
    def _():
        gacc[...] = jnp.zeros_like(gacc)
        nacc[...] = jnp.zeros_like(nacc)

    gacc[...] += jnp.broadcast_to(jnp.sum(h, axis=0, keepdims=True), (8, C))
    nacc[...] += lax.dot_general(h, h, (((0,), (0,)), ((), ())),
                                 precision=_HIGHEST,
                                 preferred_element_type=jnp.float32)

    @pl.when(j == NB - 1)
    def _():
        gb_ref[...] = gacc[...]
        nb_ref[...] = nacc[...]


def _tc_embed(x, W_emb, b_emb, Wg0):
    return pl.pallas_call(
        _tc_embed_body,
        grid=(NB,),
        in_specs=[_row_spec(), _full_spec((C, C)), _full_spec((1, C)),
                  _full_spec((C, C))],
        out_specs=[_row_spec(), _row_spec(),
                   _full_spec((8, C)), _full_spec((C, C))],
        out_shape=[jax.ShapeDtypeStruct((N, C), jnp.float32),
                   jax.ShapeDtypeStruct((N, C), jnp.float32),
                   jax.ShapeDtypeStruct((8, C), jnp.float32),
                   jax.ShapeDtypeStruct((C, C), jnp.float32)],
        scratch_shapes=[pltpu.VMEM((8, C), jnp.float32),
                        pltpu.VMEM((C, C), jnp.float32)],
    )(x, W_emb, b_emb, Wg0)


def _tc_scale_body(hl_ref, degp_ref, hs_ref, dinv8_ref):
    dinv = _dinv_block(degp_ref)
    dinv8_ref[...] = jnp.broadcast_to(dinv, (BN, 8))
    hs_ref[...] = hl_ref[...] * dinv


def _tc_scale(hl, degp):
    return pl.pallas_call(
        _tc_scale_body,
        grid=(NB,),
        in_specs=[_row_spec(), _degp_spec()],
        out_specs=[_row_spec(), pl.BlockSpec((BN, 8), lambda j: (j, 0))],
        out_shape=[jax.ShapeDtypeStruct((N, C), jnp.float32),
                   jax.ShapeDtypeStruct((N, 8), jnp.float32)],
    )(hl, degp)


_INV_NC = 1.0 / (N * C)
_EPS = 1e-5


def _dinv8_spec_ph(phases):
    def imap(p, j):
        use = (p == phases[0])
        for q in phases[1:]:
            use = use | (p == q)
        return (jnp.where(use, j, 0), 0)
    return pl.BlockSpec((BN, 8), imap)


def _row_spec_ph(phase):
    return pl.BlockSpec((BN, C), lambda p, j: (jnp.where(p == phase, j, 0), 0))


def _full_spec2(shape):
    return pl.BlockSpec(shape, lambda p, j: tuple(0 for _ in shape))


def _smem_spec2():
    return pl.BlockSpec(memory_space=pltpu.SMEM)


def _make_tc_layer_body(has_next):
    def body(*refs):
        if has_next:
            (aggp_ref, hs_ref, dinv8_ref, bg_ref, hp_ref, wf_ref, bf_ref,
             ln1w_ref, ln1b_ref, ln2w_ref, ln2b_ref, wgn_ref,
             hn_ref, hsn_ref, gb_ref, nb_ref,
             m_all, f_all, sm1, sm2, sf1, sf2, gacc, nacc) = refs
        else:
            (aggp_ref, hs_ref, dinv8_ref, bg_ref, hp_ref, wf_ref, bf_ref,
             ln1w_ref, ln1b_ref, ln2w_ref, ln2b_ref,
             gb_ref, nb_ref,
             m_all, f_all, sm1, sm2, sf1, sf2, gacc, nacc) = refs
        p = pl.program_id(0)
        j = pl.program_id(1)
        rows = pl.ds(j * BN, BN)

        @pl.when(p == 0)
        def _():
            dinv = dinv8_ref[:, 0:1]
            m = (aggp_ref[0] + aggp_ref[1] + hs_ref[...]) * dinv + bg_ref[...]
            m_all[rows, :] = m

            @pl.when(j == 0)
            def _():
                sm1[0, 0] = 0.0
                sm2[0, 0] = 0.0

            sm1[0, 0] += jnp.sum(m)
            sm2[0, 0] += jnp.sum(m * m)

        @pl.when(p == 1)
        def _():
            mean = sm1[0, 0] * _INV_NC
            var = sm2[0, 0] * _INV_NC - mean * mean
            rstd = lax.rsqrt(var + _EPS)
            mhat = ((m_all[rows, :] - mean) * rstd * ln1w_ref[0, 0]
                    + ln1b_ref[0, 0])
            hmid = hp_ref[...] + jnp.maximum(mhat, 0.0)
            f = jnp.dot(hmid, wf_ref[...], precision=_HIGHEST,
                        preferred_element_type=jnp.float32) + bf_ref[...]
            f_all[rows, :] = f

            @pl.when(j == 0)
            def _():
                sf1[0, 0] = 0.0
                sf2[0, 0] = 0.0

            sf1[0, 0] += jnp.sum(f)
            sf2[0, 0] += jnp.sum(f * f)

        @pl.when(p == 2)
        def _():
            mean = sf1[0, 0] * _INV_NC
            var = sf2[0, 0] * _INV_NC - mean * mean
            rstd = lax.rsqrt(var + _EPS)
            fhat = ((f_all[rows, :] - mean) * rstd * ln2w_ref[0, 0]
                    + ln2b_ref[0, 0])
            hn = jnp.maximum(fhat, 0.0)

            @pl.when(j == 0)
            def _():
                gacc[...] = jnp.zeros_like(gacc)
                nacc[...] = jnp.zeros_like(nacc)

            gacc[...] += jnp.broadcast_to(
                jnp.sum(hn, axis=0, keepdims=True), (8, C))
            nacc[...] += lax.dot_general(hn, hn, (((0,), (0,)), ((), ())),
                                         precision=_HIGHEST,
                                         preferred_element_type=jnp.float32)
            if has_next:
                hn_ref[...] = hn
                hl = jnp.dot(hn, wgn_ref[...], precision=_HIGHEST,
                             preferred_element_type=jnp.float32)
                hsn_ref[...] = hl * dinv8_ref[:, 0:1]

            @pl.when(j == NB - 1)
            def _():
                gb_ref[...] = gacc[...]
                nb_ref[...] = nacc[...]

    return body


def _tc_layer(aggp, hs, dinv8, bg_i, hprev, Wf_i, bf_i, ln1w, ln1b,
              ln2w, ln2b, Wg_next):
    has_next = Wg_next is not None
    in_specs = [
        pl.BlockSpec((2, BN, C), lambda p, j: (0, jnp.where(p == 0, j, 0), 0)),
        _row_spec_ph(0),
        _dinv8_spec_ph((0, 2) if has_next else (0,)),
        _full_spec2((1, C)),
        _row_spec_ph(1),
        _full_spec2((C, C)),
        _full_spec2((1, C)),
        _smem_spec2(), _smem_spec2(), _smem_spec2(), _smem_spec2(),
    ]
    args = [aggp, hs, dinv8, bg_i, hprev, Wf_i, bf_i, ln1w, ln1b, ln2w, ln2b]
    if has_next:
        in_specs.append(_full_spec2((C, C)))
        args.append(Wg_next)
        out_specs = [_row_spec_ph(2), _row_spec_ph(2),
                     _full_spec2((8, C)), _full_spec2((C, C))]
        out_shape = [jax.ShapeDtypeStruct((N, C), jnp.float32),
                     jax.ShapeDtypeStruct((N, C), jnp.float32),
                     jax.ShapeDtypeStruct((8, C), jnp.float32),
                     jax.ShapeDtypeStruct((C, C), jnp.float32)]
    else:
        out_specs = [_full_spec2((8, C)), _full_spec2((C, C))]
        out_shape = [jax.ShapeDtypeStruct((8, C), jnp.float32),
                     jax.ShapeDtypeStruct((C, C), jnp.float32)]
    return pl.pallas_call(
        _make_tc_layer_body(has_next),
        grid=(3, NB),
        in_specs=in_specs,
        out_specs=out_specs,
        out_shape=out_shape,
        scratch_shapes=[pltpu.VMEM((N, C), jnp.float32),
                        pltpu.VMEM((N, C), jnp.float32),
                        pltpu.SMEM((1, 1), jnp.float32),
                        pltpu.SMEM((1, 1), jnp.float32),
                        pltpu.SMEM((1, 1), jnp.float32),
                        pltpu.SMEM((1, 1), jnp.float32),
                        pltpu.VMEM((8, C), jnp.float32),
                        pltpu.VMEM((C, C), jnp.float32)],
    )(*args)


def kernel(x, edge_index, batch, W_emb, b_emb, Wg, bg, ln1w, ln1b,
           Wf, bf, ln2w, ln2b):
    src = edge_index[0]
    dst = edge_index[1]
    degp = jnp.broadcast_to(_sc_deg(dst).reshape(2, N, 1), (2, N, 8))

    h0, hl0, gb0, nb0 = _tc_embed(x, W_emb, b_emb[None, :], Wg[0])
    hs0, dinv8 = _tc_scale(hl0, degp)

    gbs, nbs = [gb0], [nb0]
    h, hs = h0, hs0
    for i in range(L):
        aggp = _sc_agg(src, dst, hs).reshape(2, N, C)
        outs = _tc_layer(aggp, hs, dinv8, bg[i][None, :], h,
                         Wf[i], bf[i][None, :],
                         ln1w[i].reshape(1, 1), ln1b[i].reshape(1, 1),
                         ln2w[i].reshape(1, 1), ln2b[i].reshape(1, 1),
                         Wg[i + 1] if i + 1 < L else None)
        if i + 1 < L:
            h, hs, gb_i, nb_i = outs
        else:
            gb_i, nb_i = outs
        gbs.append(gb_i)
        nbs.append(nb_i)

    gb_out = jnp.concatenate([g[0:1] for g in gbs], axis=-1)
    nb_out = jnp.stack(nbs, axis=0)[None]
    return gb_out, nb_out
